# Initial kernel scaffold; baseline (speedup 1.0000x reference)
import jax
import jax.numpy as jnp
from jax.experimental import pallas as pl


def kernel(l_data, edge_index, W0, b0, W1, b1, W2, b2):
    raise NotImplementedError("write your pallas kernel here")


# speedup vs baseline: 6.2591x; 6.2591x over previous
import functools

import jax
import jax.numpy as jnp
from jax import lax
from jax.experimental import pallas as pl
from jax.experimental.pallas import tpu as pltpu
from jax.experimental.pallas import tpu_sc as plsc

N = 100000
E = 1600000
HID = 32
HHID = HID // 2
NSUB = 16
NCORE = 2
NP = 102400
RPT = NP // NSUB
CH = 128
NCHUNK = E // CH
CPS = NCHUNK // NCORE
ZR = RPT // 8

_MESH = plsc.VectorSubcoreMesh(core_axis_name="c", subcore_axis_name="s")
_SC_PARAMS = pltpu.CompilerParams(use_tc_tiling_on_sc=False)
_INTERP = False


def _hist_body(src_hbm, dst_hbm, z1_hbm, ones_hbm,
               od_out, id_out,
               od_sh, id_sh, sidx, didx, onesv, zbuf):
    c = lax.axis_index("c")
    s = lax.axis_index("s")
    r0 = s * RPT
    pltpu.sync_copy(ones_hbm, onesv)
    pltpu.sync_copy(z1_hbm, zbuf)
    pltpu.sync_copy(zbuf, od_sh.at[pl.ds(r0, RPT)])
    pltpu.sync_copy(zbuf, id_sh.at[pl.ds(r0, RPT)])
    plsc.subcore_barrier()

    nk = 390 + jnp.where(s < 10, 1, 0)

    def body(k, _):
        cc = c * CPS + s + NSUB * k
        off = cc * CH
        pltpu.sync_copy(src_hbm.at[pl.ds(off, CH)], sidx)
        pltpu.sync_copy(dst_hbm.at[pl.ds(off, CH)], didx)
        pltpu.sync_copy(onesv, od_sh.at[sidx], add=True)
        pltpu.sync_copy(onesv, id_sh.at[didx], add=True)
        return 0

    lax.fori_loop(0, nk, body, 0)
    plsc.subcore_barrier()

    o0 = c * NP + r0
    pltpu.sync_copy(od_sh.at[pl.ds(r0, RPT)], zbuf)
    pltpu.sync_copy(zbuf, od_out.at[pl.ds(o0, RPT)])
    pltpu.sync_copy(id_sh.at[pl.ds(r0, RPT)], zbuf)
    pltpu.sync_copy(zbuf, id_out.at[pl.ds(o0, RPT)])


_hist = functools.partial(
    pl.kernel,
    out_type=(jax.ShapeDtypeStruct((NCORE * NP,), jnp.float32),
              jax.ShapeDtypeStruct((NCORE * NP,), jnp.float32)),
    mesh=_MESH,
    scratch_types=[
        pltpu.VMEM_SHARED((NP,), jnp.float32),
        pltpu.VMEM_SHARED((NP,), jnp.float32),
        pltpu.VMEM((CH,), jnp.int32),
        pltpu.VMEM((CH,), jnp.int32),
        pltpu.VMEM((CH,), jnp.float32),
        pltpu.VMEM((RPT,), jnp.float32),
    ],
    compiler_params=_SC_PARAMS,
    interpret=_INTERP,
)(_hist_body)


def _l0_body(src_hbm, dst_hbm, y0_hbm, z1_hbm,
             agg0_out,
             agg_sh, sidx, didx, vals, zbuf, sem):
    c = lax.axis_index("c")
    s = lax.axis_index("s")
    r0 = s * RPT
    pltpu.sync_copy(z1_hbm, zbuf)
    pltpu.sync_copy(zbuf, agg_sh.at[pl.ds(r0, RPT)])
    plsc.subcore_barrier()

    nk = 390 + jnp.where(s < 10, 1, 0)

    def body(k, _):
        cc = c * CPS + s + NSUB * k
        off = cc * CH
        pltpu.sync_copy(src_hbm.at[pl.ds(off, CH)], sidx)
        pltpu.sync_copy(dst_hbm.at[pl.ds(off, CH)], didx)
        pltpu.async_copy(y0_hbm.at[sidx], vals, sem).wait()
        pltpu.sync_copy(vals, agg_sh.at[didx], add=True)
        return 0

    lax.fori_loop(0, nk, body, 0)
    plsc.subcore_barrier()

    pltpu.sync_copy(agg_sh.at[pl.ds(r0, RPT)], zbuf)
    pltpu.sync_copy(zbuf, agg0_out.at[pl.ds(c * NP + r0, RPT)])


_l0 = functools.partial(
    pl.kernel,
    out_type=jax.ShapeDtypeStruct((NCORE * NP,), jnp.float32),
    mesh=_MESH,
    scratch_types=[
        pltpu.VMEM_SHARED((NP,), jnp.float32),
        pltpu.VMEM((CH,), jnp.int32),
        pltpu.VMEM((CH,), jnp.int32),
        pltpu.VMEM((CH,), jnp.float32),
        pltpu.VMEM((RPT,), jnp.float32),
        pltpu.SemaphoreType.DMA,
    ],
    compiler_params=_SC_PARAMS,
    interpret=_INTERP,
)(_l0_body)


def _conv_body(src_hbm, dst_hbm, x_hbm, z16_hbm,
               agg_out,
               agg_sh, sidx, didx, vals, zbuf, sem):
    c = lax.axis_index("c")
    s = lax.axis_index("s")
    r0 = s * RPT
    coff = c * NP
    pltpu.sync_copy(z16_hbm, zbuf)
    for r in range(8):
        pltpu.sync_copy(zbuf, agg_sh.at[pl.ds(r0 + r * ZR, ZR)])
    plsc.subcore_barrier()

    nk = 781 + jnp.where(s < 4, 1, 0)

    def body(k, _):
        cc = s + NSUB * k
        off = cc * CH
        pltpu.sync_copy(src_hbm.at[pl.ds(off, CH)], sidx)
        pltpu.sync_copy(dst_hbm.at[pl.ds(off, CH)], didx)
        for j in range(CH // 16):
            sidx[pl.ds(j * 16, 16)] = sidx[pl.ds(j * 16, 16)] + coff
        pltpu.async_copy(x_hbm.at[sidx], vals, sem).wait()
        pltpu.sync_copy(vals, agg_sh.at[didx], add=True)
        return 0

    lax.fori_loop(0, nk, body, 0)
    plsc.subcore_barrier()

    for r in range(8):
        pltpu.sync_copy(agg_sh.at[pl.ds(r0 + r * ZR, ZR)], zbuf)
        pltpu.sync_copy(zbuf, agg_out.at[pl.ds(coff + r0 + r * ZR, ZR)])


_conv = functools.partial(
    pl.kernel,
    out_type=jax.ShapeDtypeStruct((NCORE * NP, HHID), jnp.float32),
    mesh=_MESH,
    scratch_types=[
        pltpu.VMEM_SHARED((NP, HHID), jnp.float32),
        pltpu.VMEM((CH,), jnp.int32),
        pltpu.VMEM((CH,), jnp.int32),
        pltpu.VMEM((CH, HHID), jnp.float32),
        pltpu.VMEM((ZR, HHID), jnp.float32),
        pltpu.SemaphoreType.DMA,
    ],
    compiler_params=_SC_PARAMS,
    interpret=_INTERP,
)(_conv_body)


_RB = RPT
_G = NP // _RB

_col1 = pl.BlockSpec((_RB, 1), lambda i: (i, 0))
_col1b = pl.BlockSpec((_RB, 1), lambda i: (i + _G, 0))
_colH = pl.BlockSpec((_RB, HID), lambda i: (i, 0))
_colHH = pl.BlockSpec((_RB, HHID), lambda i: (i, 0))
_colHHb = pl.BlockSpec((_RB, HHID), lambda i: (i + _G, 0))
_wfull = pl.BlockSpec((HID, HID), lambda i: (0, 0))
_w0full = pl.BlockSpec((1, HID), lambda i: (0, 0))


def _norm_body(oda_ref, odb_ref, ida_ref, idb_ref, l_ref,
               ns_ref, nd_ref, y0_ref):
    od = oda_ref[...] + odb_ref[...]
    idg = ida_ref[...] + idb_ref[...]
    ns = jnp.where(od > 0, lax.rsqrt(jnp.maximum(od, 1.0)), 0.0)
    nd = jnp.where(idg > 0, lax.rsqrt(jnp.maximum(idg, 1.0)), 0.0)
    ns_ref[...] = ns
    nd_ref[...] = nd
    y0_ref[...] = l_ref[...] * ns


_norms = pl.pallas_call(
    _norm_body,
    grid=(_G,),
    in_specs=[_col1, _col1b, _col1, _col1b, _col1],
    out_specs=[_col1, _col1, _col1],
    out_shape=[jax.ShapeDtypeStruct((NP, 1), jnp.float32)] * 3,
    interpret=_INTERP,
)


def _dense0_body(a_ref, a2_ref, nd_ref, ns_ref, w_ref, b_ref,
                 xa_ref, xb_ref):
    a = a_ref[...] + a2_ref[...]
    h = a * nd_ref[...]
    out = h * w_ref[...] + b_ref[...]
    out = jnp.maximum(out, 0.0) * ns_ref[...]
    xa_ref[...] = out[:, :HHID]
    xb_ref[...] = out[:, HHID:]


_dense0 = pl.pallas_call(
    _dense0_body,
    grid=(_G,),
    in_specs=[_col1, _col1b, _col1, _col1, _w0full, _w0full],
    out_specs=[_colHH, _colHH],
    out_shape=[jax.ShapeDtypeStruct((NP, HHID), jnp.float32)] * 2,
    interpret=_INTERP,
)


def _dense_body(aa_ref, ab_ref, nd_ref, ns_ref, w_ref, b_ref, xa_ref, xb_ref):
    agg = jnp.concatenate([aa_ref[...], ab_ref[...]], axis=1)
    h = jnp.dot(agg, w_ref[...], preferred_element_type=jnp.float32)
    out = jnp.maximum(h * nd_ref[...] + b_ref[...], 0.0) * ns_ref[...]
    xa_ref[...] = out[:, :HHID]
    xb_ref[...] = out[:, HHID:]


_dense = pl.pallas_call(
    _dense_body,
    grid=(_G,),
    in_specs=[_colHH, _colHHb, _col1, _col1, _wfull, _w0full],
    out_specs=[_colHH, _colHH],
    out_shape=[jax.ShapeDtypeStruct((NP, HHID), jnp.float32)] * 2,
    interpret=_INTERP,
)


def _final_body(aa_ref, ab_ref, nd_ref, w_ref, b_ref, out_ref):
    agg = jnp.concatenate([aa_ref[...], ab_ref[...]], axis=1)
    h = jnp.dot(agg, w_ref[...], preferred_element_type=jnp.float32)
    out_ref[...] = h * nd_ref[...] + b_ref[...]


_final = pl.pallas_call(
    _final_body,
    grid=(_G,),
    in_specs=[_colHH, _colHHb, _col1, _wfull, _w0full],
    out_specs=_colH,
    out_shape=jax.ShapeDtypeStruct((NP, HID), jnp.float32),
    interpret=_INTERP,
)


def kernel(l_data, edge_index, W0, b0, W1, b1, W2, b2):
    src = edge_index[0]
    dst = edge_index[1]
    l_pad = jnp.pad(l_data, ((0, NP - N), (0, 0)))

    z1 = jnp.zeros((RPT,), jnp.float32)
    z16 = jnp.zeros((ZR, HHID), jnp.float32)
    ones = jnp.ones((CH,), jnp.float32)

    odp, idp = _hist(src, dst, z1, ones)
    odp = odp.reshape(NCORE * NP, 1)
    idp = idp.reshape(NCORE * NP, 1)
    ns, nd, y0 = _norms(odp, odp, idp, idp, l_pad)

    agg0 = _l0(src, dst, y0.reshape(NP), z1).reshape(NCORE * NP, 1)

    xa, xb = _dense0(agg0, agg0, nd, ns, W0.reshape(1, HID), b0.reshape(1, HID))
    agg = _conv(src, dst, jnp.concatenate([xa, xb], axis=0), z16)
    xa, xb = _dense(agg, agg, nd, ns, W1, b1.reshape(1, HID))
    agg = _conv(src, dst, jnp.concatenate([xa, xb], axis=0), z16)
    out = _final(agg, agg, nd, W2, b2.reshape(1, HID))
    return out[:N]


# speedup vs baseline: 15.2755x; 2.4405x over previous
import functools

import jax
import jax.numpy as jnp
from jax import lax
from jax.experimental import pallas as pl
from jax.experimental.pallas import tpu as pltpu
from jax.experimental.pallas import tpu_sc as plsc

N = 100000
E = 1600000
HID = 32
HHID = HID // 2
NSUB = 16
NCORE = 2
NW = NCORE * NSUB
NP = 102400
RPT = NP // NSUB
CH = 128
NB = 4
NCH = 12544
E2 = NCH * CH
CPT = NCH // NSUB
CPW = NCH // NW
ZR = 400
NZ = RPT // ZR

_MESH = plsc.VectorSubcoreMesh(core_axis_name="c", subcore_axis_name="s")
_SC_PARAMS = pltpu.CompilerParams(use_tc_tiling_on_sc=False)
_INTERP = False


def _hist_body(earr, z1_hbm, ones_hbm,
               od_out, id_out,
               od_sh, id_sh, i0, i1, onesv, zbuf,
               sem_i0, sem_i1, sem_s0, sem_s1):
    c = lax.axis_index("c")
    s = lax.axis_index("s")
    r0 = s * RPT
    w = c * NSUB + s
    base = w * CPW
    ib = (i0, i1)
    sem_i = (sem_i0, sem_i1)
    sem_s = (sem_s0, sem_s1)

    pltpu.sync_copy(ones_hbm, onesv)
    pltpu.sync_copy(z1_hbm, zbuf)
    pltpu.sync_copy(zbuf, od_sh.at[pl.ds(r0, RPT)])
    pltpu.sync_copy(zbuf, id_sh.at[pl.ds(r0, RPT)])
    plsc.subcore_barrier()

    pltpu.async_copy(earr.at[pl.ds(base, NB)], i0, sem_i0)

    def group(p, g, first):
        ip = sem_i[p]
        pltpu.make_async_copy(earr.at[pl.ds(0, NB)], ib[p], ip).wait()
        if first:
            @pl.when(g >= 1)
            def _():
                pltpu.make_async_copy(ones_hbm, onesv, sem_s[1 - p]).wait()
        else:
            pltpu.make_async_copy(ones_hbm, onesv, sem_s[1 - p]).wait()
        pltpu.async_copy(earr.at[pl.ds(base + NB * (g + 1), NB)],
                         ib[1 - p], sem_i[1 - p])
        for b in range(NB):
            pltpu.async_copy(onesv.at[0], od_sh.at[ib[p].at[b, 0]],
                             sem_s[p], add=True)
            pltpu.async_copy(onesv.at[0], id_sh.at[ib[p].at[b, 1]],
                             sem_s[p], add=True)

    def body(gg, _):
        group(0, 2 * gg, True)
        group(1, 2 * gg + 1, False)
        return 0

    lax.fori_loop(0, CPW // NB // 2, body, 0)
    pltpu.make_async_copy(ones_hbm, onesv, sem_s1).wait()
    pltpu.make_async_copy(earr.at[pl.ds(0, NB)], i0, sem_i0).wait()
    plsc.subcore_barrier()

    o0 = c * NP + r0
    pltpu.sync_copy(od_sh.at[pl.ds(r0, RPT)], zbuf)
    pltpu.sync_copy(zbuf, od_out.at[pl.ds(o0, RPT)])
    pltpu.sync_copy(id_sh.at[pl.ds(r0, RPT)], zbuf)
    pltpu.sync_copy(zbuf, id_out.at[pl.ds(o0, RPT)])


_hist = functools.partial(
    pl.kernel,
    out_type=(jax.ShapeDtypeStruct((NCORE * NP,), jnp.float32),
              jax.ShapeDtypeStruct((NCORE * NP,), jnp.float32)),
    mesh=_MESH,
    scratch_types=[
        pltpu.VMEM_SHARED((NP,), jnp.float32),
        pltpu.VMEM_SHARED((NP,), jnp.float32),
        pltpu.VMEM((NB, 2, CH), jnp.int32),
        pltpu.VMEM((NB, 2, CH), jnp.int32),
        pltpu.VMEM((2 * NB, CH), jnp.float32),
        pltpu.VMEM((RPT,), jnp.float32),
        pltpu.SemaphoreType.DMA,
        pltpu.SemaphoreType.DMA,
        pltpu.SemaphoreType.DMA,
        pltpu.SemaphoreType.DMA,
    ],
    compiler_params=_SC_PARAMS,
    interpret=_INTERP,
)(_hist_body)


def _l0_body(earr, y0_hbm, z1_hbm,
             agg0_out,
             agg_sh, i0, i1, v0, v1, zbuf,
             sem_i0, sem_i1, sem_g0, sem_g1, sem_s0, sem_s1):
    c = lax.axis_index("c")
    s = lax.axis_index("s")
    r0 = s * RPT
    w = c * NSUB + s
    base = w * CPW
    ib = (i0, i1)
    vb = (v0, v1)
    sem_i = (sem_i0, sem_i1)
    sem_g = (sem_g0, sem_g1)
    sem_s = (sem_s0, sem_s1)

    pltpu.sync_copy(z1_hbm, zbuf)
    pltpu.sync_copy(zbuf, agg_sh.at[pl.ds(r0, RPT)])
    plsc.subcore_barrier()

    pltpu.async_copy(earr.at[pl.ds(base, NB)], i0, sem_i0)

    def group(p, g, first):
        pltpu.make_async_copy(earr.at[pl.ds(0, NB)], ib[p], sem_i[p]).wait()
        if first:
            @pl.when(g >= 1)
            def _():
                pltpu.make_async_copy(y0_hbm.at[pl.ds(0, NB * CH)],
                                      vb[1 - p], sem_s[1 - p]).wait()
        else:
            pltpu.make_async_copy(y0_hbm.at[pl.ds(0, NB * CH)],
                                  vb[1 - p], sem_s[1 - p]).wait()
        pltpu.async_copy(earr.at[pl.ds(base + NB * (g + 1), NB)],
                         ib[1 - p], sem_i[1 - p])
        for b in range(NB):
            pltpu.async_copy(y0_hbm.at[ib[p].at[b, 0]],
                             vb[p].at[pl.ds(b * CH, CH)], sem_g[p])
        pltpu.make_async_copy(y0_hbm.at[pl.ds(0, NB * CH)],
                              vb[p], sem_g[p]).wait()
        for b in range(NB):
            pltpu.async_copy(vb[p].at[pl.ds(b * CH, CH)],
                             agg_sh.at[ib[p].at[b, 1]], sem_s[p], add=True)

    def body(gg, _):
        group(0, 2 * gg, True)
        group(1, 2 * gg + 1, False)
        return 0

    lax.fori_loop(0, CPW // NB // 2, body, 0)
    pltpu.make_async_copy(y0_hbm.at[pl.ds(0, NB * CH)], v1, sem_s1).wait()
    pltpu.make_async_copy(earr.at[pl.ds(0, NB)], i0, sem_i0).wait()
    plsc.subcore_barrier()

    pltpu.sync_copy(agg_sh.at[pl.ds(r0, RPT)], zbuf)
    pltpu.sync_copy(zbuf, agg0_out.at[pl.ds(c * NP + r0, RPT)])


_l0 = functools.partial(
    pl.kernel,
    out_type=jax.ShapeDtypeStruct((NCORE * NP,), jnp.float32),
    mesh=_MESH,
    scratch_types=[
        pltpu.VMEM_SHARED((NP,), jnp.float32),
        pltpu.VMEM((NB, 2, CH), jnp.int32),
        pltpu.VMEM((NB, 2, CH), jnp.int32),
        pltpu.VMEM((NB * CH,), jnp.float32),
        pltpu.VMEM((NB * CH,), jnp.float32),
        pltpu.VMEM((RPT,), jnp.float32),
        pltpu.SemaphoreType.DMA,
        pltpu.SemaphoreType.DMA,
        pltpu.SemaphoreType.DMA,
        pltpu.SemaphoreType.DMA,
        pltpu.SemaphoreType.DMA,
        pltpu.SemaphoreType.DMA,
    ],
    compiler_params=_SC_PARAMS,
    interpret=_INTERP,
)(_l0_body)


def _conv_body(earr, x_hbm, z16_hbm,
               agg_out,
               agg_sh, i0, i1, v0, v1, zbuf,
               sem_i0, sem_i1, sem_g0, sem_g1, sem_s0, sem_s1):
    c = lax.axis_index("c")
    s = lax.axis_index("s")
    r0 = s * RPT
    coff = c * NP
    base = s * CPT
    ib = (i0, i1)
    vb = (v0, v1)
    sem_i = (sem_i0, sem_i1)
    sem_g = (sem_g0, sem_g1)
    sem_s = (sem_s0, sem_s1)

    pltpu.sync_copy(z16_hbm, zbuf)
    for r in range(NZ):
        pltpu.async_copy(zbuf, agg_sh.at[pl.ds(r0 + r * ZR, ZR)], sem_g0)
    for r in range(NZ):
        pltpu.make_async_copy(zbuf, agg_sh.at[pl.ds(r0 + r * ZR, ZR)],
                              sem_g0).wait()
    plsc.subcore_barrier()

    pltpu.async_copy(earr.at[pl.ds(base, NB)], i0, sem_i0)

    def group(p, g, first):
        pltpu.make_async_copy(earr.at[pl.ds(0, NB)], ib[p], sem_i[p]).wait()
        if first:
            @pl.when(g >= 1)
            def _():
                pltpu.make_async_copy(x_hbm.at[pl.ds(0, NB * CH)],
                                      vb[1 - p], sem_s[1 - p]).wait()
        else:
            pltpu.make_async_copy(x_hbm.at[pl.ds(0, NB * CH)],
                                  vb[1 - p], sem_s[1 - p]).wait()
        pltpu.async_copy(earr.at[pl.ds(base + NB * (g + 1), NB)],
                         ib[1 - p], sem_i[1 - p])
        for b in range(NB):
            for j in range(CH // 16):
                ib[p][b, 0, pl.ds(j * 16, 16)] = (
                    ib[p][b, 0, pl.ds(j * 16, 16)] + coff)
        for b in range(NB):
            pltpu.async_copy(x_hbm.at[ib[p].at[b, 0]],
                             vb[p].at[pl.ds(b * CH, CH)], sem_g[p])
        pltpu.make_async_copy(x_hbm.at[pl.ds(0, NB * CH)],
                              vb[p], sem_g[p]).wait()
        for b in range(NB):
            pltpu.async_copy(vb[p].at[pl.ds(b * CH, CH)],
                             agg_sh.at[ib[p].at[b, 1]], sem_s[p], add=True)

    def body(gg, _):
        group(0, 2 * gg, True)
        group(1, 2 * gg + 1, False)
        return 0

    lax.fori_loop(0, CPT // NB // 2, body, 0)
    pltpu.make_async_copy(x_hbm.at[pl.ds(0, NB * CH)], v1, sem_s1).wait()
    pltpu.make_async_copy(earr.at[pl.ds(0, NB)], i0, sem_i0).wait()
    plsc.subcore_barrier()

    for r in range(NZ):
        if r % 2 == 0:
            buf = zbuf
            semx = sem_g0
        else:
            buf = v1
            semx = sem_g1
        if r >= 2:
            pltpu.make_async_copy(x_hbm.at[pl.ds(0, ZR)],
                                  buf if r % 2 == 0 else buf.at[pl.ds(0, ZR)],
                                  semx).wait()
        piece = pl.ds(r0 + r * ZR, ZR)
        if r % 2 == 0:
            pltpu.sync_copy(agg_sh.at[piece], buf)
            pltpu.async_copy(buf, agg_out.at[pl.ds(coff + r0 + r * ZR, ZR)],
                             semx)
        else:
            pltpu.sync_copy(agg_sh.at[piece], buf.at[pl.ds(0, ZR)])
            pltpu.async_copy(buf.at[pl.ds(0, ZR)],
                             agg_out.at[pl.ds(coff + r0 + r * ZR, ZR)], semx)
    pltpu.make_async_copy(x_hbm.at[pl.ds(0, ZR)], zbuf, sem_g0).wait()
    pltpu.make_async_copy(x_hbm.at[pl.ds(0, ZR)], v1.at[pl.ds(0, ZR)],
                          sem_g1).wait()


_conv = functools.partial(
    pl.kernel,
    out_type=jax.ShapeDtypeStruct((NCORE * NP, HHID), jnp.float32),
    mesh=_MESH,
    scratch_types=[
        pltpu.VMEM_SHARED((NP, HHID), jnp.float32),
        pltpu.VMEM((NB, 2, CH), jnp.int32),
        pltpu.VMEM((NB, 2, CH), jnp.int32),
        pltpu.VMEM((NB * CH, HHID), jnp.float32),
        pltpu.VMEM((NB * CH, HHID), jnp.float32),
        pltpu.VMEM((ZR, HHID), jnp.float32),
        pltpu.SemaphoreType.DMA,
        pltpu.SemaphoreType.DMA,
        pltpu.SemaphoreType.DMA,
        pltpu.SemaphoreType.DMA,
        pltpu.SemaphoreType.DMA,
        pltpu.SemaphoreType.DMA,
    ],
    compiler_params=_SC_PARAMS,
    interpret=_INTERP,
)(_conv_body)


_RB = RPT
_G = NP // _RB

_col1 = pl.BlockSpec((_RB, 1), lambda i: (i, 0))
_col1b = pl.BlockSpec((_RB, 1), lambda i: (i + _G, 0))
_colH = pl.BlockSpec((_RB, HID), lambda i: (i, 0))
_colHH = pl.BlockSpec((_RB, HHID), lambda i: (i, 0))
_colHHb = pl.BlockSpec((_RB, HHID), lambda i: (i + _G, 0))
_wfull = pl.BlockSpec((HID, HID), lambda i: (0, 0))
_w0full = pl.BlockSpec((1, HID), lambda i: (0, 0))


def _norm_body(oda_ref, odb_ref, ida_ref, idb_ref, l_ref,
               ns_ref, nd_ref, y0_ref):
    od = oda_ref[...] + odb_ref[...]
    idg = ida_ref[...] + idb_ref[...]
    ns = jnp.where(od > 0, lax.rsqrt(jnp.maximum(od, 1.0)), 0.0)
    nd = jnp.where(idg > 0, lax.rsqrt(jnp.maximum(idg, 1.0)), 0.0)
    ns_ref[...] = ns
    nd_ref[...] = nd
    y0_ref[...] = l_ref[...] * ns


_norms = pl.pallas_call(
    _norm_body,
    grid=(_G,),
    in_specs=[_col1, _col1b, _col1, _col1b, _col1],
    out_specs=[_col1, _col1, _col1],
    out_shape=[jax.ShapeDtypeStruct((NP, 1), jnp.float32)] * 3,
    interpret=_INTERP,
)


def _dense0_body(a_ref, a2_ref, nd_ref, ns_ref, w_ref, b_ref,
                 xa_ref, xb_ref):
    a = a_ref[...] + a2_ref[...]
    h = a * nd_ref[...]
    out = h * w_ref[...] + b_ref[...]
    out = jnp.maximum(out, 0.0) * ns_ref[...]
    xa_ref[...] = out[:, :HHID]
    xb_ref[...] = out[:, HHID:]


_dense0 = pl.pallas_call(
    _dense0_body,
    grid=(_G,),
    in_specs=[_col1, _col1b, _col1, _col1, _w0full, _w0full],
    out_specs=[_colHH, _colHH],
    out_shape=[jax.ShapeDtypeStruct((NP, HHID), jnp.float32)] * 2,
    interpret=_INTERP,
)


def _dense_body(aa_ref, ab_ref, nd_ref, ns_ref, w_ref, b_ref, xa_ref, xb_ref):
    agg = jnp.concatenate([aa_ref[...], ab_ref[...]], axis=1)
    h = jnp.dot(agg, w_ref[...], preferred_element_type=jnp.float32)
    out = jnp.maximum(h * nd_ref[...] + b_ref[...], 0.0) * ns_ref[...]
    xa_ref[...] = out[:, :HHID]
    xb_ref[...] = out[:, HHID:]


_dense = pl.pallas_call(
    _dense_body,
    grid=(_G,),
    in_specs=[_colHH, _colHHb, _col1, _col1, _wfull, _w0full],
    out_specs=[_colHH, _colHH],
    out_shape=[jax.ShapeDtypeStruct((NP, HHID), jnp.float32)] * 2,
    interpret=_INTERP,
)


def _final_body(aa_ref, ab_ref, nd_ref, w_ref, b_ref, out_ref):
    agg = jnp.concatenate([aa_ref[...], ab_ref[...]], axis=1)
    h = jnp.dot(agg, w_ref[...], preferred_element_type=jnp.float32)
    out_ref[...] = h * nd_ref[...] + b_ref[...]


_final = pl.pallas_call(
    _final_body,
    grid=(_G,),
    in_specs=[_colHH, _colHHb, _col1, _wfull, _w0full],
    out_specs=_colH,
    out_shape=jax.ShapeDtypeStruct((NP, HID), jnp.float32),
    interpret=_INTERP,
)


def kernel(l_data, edge_index, W0, b0, W1, b1, W2, b2):
    pad = jnp.full((E2 - E,), NP - 1, jnp.int32)
    srcp = jnp.concatenate([edge_index[0], pad]).reshape(NCH, 1, CH)
    dstp = jnp.concatenate([edge_index[1], pad]).reshape(NCH, 1, CH)
    earr = jnp.concatenate([srcp, dstp], axis=1)
    earr = jnp.pad(earr, ((0, NB), (0, 0), (0, 0)))
    l_pad = jnp.pad(l_data, ((0, NP - N), (0, 0)))

    z1 = jnp.zeros((RPT,), jnp.float32)
    z16 = jnp.zeros((ZR, HHID), jnp.float32)
    ones = jnp.ones((2 * NB, CH), jnp.float32)

    odp, idp = _hist(earr, z1, ones)
    odp = odp.reshape(NCORE * NP, 1)
    idp = idp.reshape(NCORE * NP, 1)
    ns, nd, y0 = _norms(odp, odp, idp, idp, l_pad)

    agg0 = _l0(earr, y0.reshape(NP), z1).reshape(NCORE * NP, 1)

    xa, xb = _dense0(agg0, agg0, nd, ns, W0.reshape(1, HID), b0.reshape(1, HID))
    agg = _conv(earr, jnp.concatenate([xa, xb], axis=0), z16)
    xa, xb = _dense(agg, agg, nd, ns, W1, b1.reshape(1, HID))
    agg = _conv(earr, jnp.concatenate([xa, xb], axis=0), z16)
    out = _final(agg, agg, nd, W2, b2.reshape(1, HID))
    return out[:N]


# speedup vs baseline: 15.4966x; 1.0145x over previous
import functools

import jax
import jax.numpy as jnp
from jax import lax
from jax.experimental import pallas as pl
from jax.experimental.pallas import tpu as pltpu
from jax.experimental.pallas import tpu_sc as plsc

N = 100000
E = 1600000
HID = 32
HHID = HID // 2
NSUB = 16
NCORE = 2
NW = NCORE * NSUB
NP = 102400
RPT = NP // NSUB
CH = 128
NB = 4
NCH = 12544
E2 = NCH * CH
CPT = NCH // NSUB
CPW = NCH // NW
ZR = 400
NZ = RPT // ZR

_MESH = plsc.VectorSubcoreMesh(core_axis_name="c", subcore_axis_name="s")
_SC_PARAMS = pltpu.CompilerParams(use_tc_tiling_on_sc=False)
_INTERP = False


def _hist_body(earr, z1_hbm, ones_hbm,
               od_out, id_out,
               od_sh, id_sh, i0, i1, onesv, zbuf,
               sem_i0, sem_i1, sem_s0, sem_s1):
    c = lax.axis_index("c")
    s = lax.axis_index("s")
    r0 = s * RPT
    w = c * NSUB + s
    base = w * CPW
    ib = (i0, i1)
    sem_i = (sem_i0, sem_i1)
    sem_s = (sem_s0, sem_s1)

    pltpu.sync_copy(ones_hbm, onesv)
    pltpu.sync_copy(z1_hbm, zbuf)
    pltpu.sync_copy(zbuf, od_sh.at[pl.ds(r0, RPT)])
    pltpu.sync_copy(zbuf, id_sh.at[pl.ds(r0, RPT)])
    plsc.subcore_barrier()

    pltpu.async_copy(earr.at[pl.ds(base, NB)], i0, sem_i0)

    def group(p, g, first):
        ip = sem_i[p]
        pltpu.make_async_copy(earr.at[pl.ds(0, NB)], ib[p], ip).wait()
        if first:
            @pl.when(g >= 1)
            def _():
                pltpu.make_async_copy(ones_hbm, onesv, sem_s[1 - p]).wait()
        else:
            pltpu.make_async_copy(ones_hbm, onesv, sem_s[1 - p]).wait()
        pltpu.async_copy(earr.at[pl.ds(base + NB * (g + 1), NB)],
                         ib[1 - p], sem_i[1 - p])
        for b in range(NB):
            pltpu.async_copy(onesv.at[0], od_sh.at[ib[p].at[b, 0]],
                             sem_s[p], add=True)
            pltpu.async_copy(onesv.at[0], id_sh.at[ib[p].at[b, 1]],
                             sem_s[p], add=True)

    def body(gg, _):
        group(0, 2 * gg, True)
        group(1, 2 * gg + 1, False)
        return 0

    lax.fori_loop(0, CPW // NB // 2, body, 0)
    pltpu.make_async_copy(ones_hbm, onesv, sem_s1).wait()
    pltpu.make_async_copy(earr.at[pl.ds(0, NB)], i0, sem_i0).wait()
    plsc.subcore_barrier()

    o0 = c * NP + r0
    pltpu.sync_copy(od_sh.at[pl.ds(r0, RPT)], zbuf)
    pltpu.sync_copy(zbuf, od_out.at[pl.ds(o0, RPT)])
    pltpu.sync_copy(id_sh.at[pl.ds(r0, RPT)], zbuf)
    pltpu.sync_copy(zbuf, id_out.at[pl.ds(o0, RPT)])


_hist = functools.partial(
    pl.kernel,
    out_type=(jax.ShapeDtypeStruct((NCORE * NP,), jnp.float32),
              jax.ShapeDtypeStruct((NCORE * NP,), jnp.float32)),
    mesh=_MESH,
    scratch_types=[
        pltpu.VMEM_SHARED((NP,), jnp.float32),
        pltpu.VMEM_SHARED((NP,), jnp.float32),
        pltpu.VMEM((NB, 2, CH), jnp.int32),
        pltpu.VMEM((NB, 2, CH), jnp.int32),
        pltpu.VMEM((2 * NB, CH), jnp.float32),
        pltpu.VMEM((RPT,), jnp.float32),
        pltpu.SemaphoreType.DMA,
        pltpu.SemaphoreType.DMA,
        pltpu.SemaphoreType.DMA,
        pltpu.SemaphoreType.DMA,
    ],
    compiler_params=_SC_PARAMS,
    interpret=_INTERP,
)(_hist_body)


def _l0_body(earr, y0_hbm, z1_hbm,
             agg0_out,
             agg_sh, i0, i1, v0, v1, zbuf,
             sem_i0, sem_i1, sem_g0, sem_g1, sem_s0, sem_s1):
    c = lax.axis_index("c")
    s = lax.axis_index("s")
    r0 = s * RPT
    w = c * NSUB + s
    base = w * CPW
    ib = (i0, i1)
    vb = (v0, v1)
    sem_i = (sem_i0, sem_i1)
    sem_g = (sem_g0, sem_g1)
    sem_s = (sem_s0, sem_s1)

    pltpu.sync_copy(z1_hbm, zbuf)
    pltpu.sync_copy(zbuf, agg_sh.at[pl.ds(r0, RPT)])
    plsc.subcore_barrier()

    pltpu.async_copy(earr.at[pl.ds(base, NB)], i0, sem_i0)

    def group(p, g, first):
        pltpu.make_async_copy(earr.at[pl.ds(0, NB)], ib[p], sem_i[p]).wait()
        if first:
            @pl.when(g >= 1)
            def _():
                pltpu.make_async_copy(y0_hbm.at[pl.ds(0, NB * CH)],
                                      vb[1 - p], sem_s[1 - p]).wait()
        else:
            pltpu.make_async_copy(y0_hbm.at[pl.ds(0, NB * CH)],
                                  vb[1 - p], sem_s[1 - p]).wait()
        pltpu.async_copy(earr.at[pl.ds(base + NB * (g + 1), NB)],
                         ib[1 - p], sem_i[1 - p])
        for b in range(NB):
            pltpu.async_copy(y0_hbm.at[ib[p].at[b, 0]],
                             vb[p].at[pl.ds(b * CH, CH)], sem_g[p])
        pltpu.make_async_copy(y0_hbm.at[pl.ds(0, NB * CH)],
                              vb[p], sem_g[p]).wait()
        for b in range(NB):
            pltpu.async_copy(vb[p].at[pl.ds(b * CH, CH)],
                             agg_sh.at[ib[p].at[b, 1]], sem_s[p], add=True)

    def body(gg, _):
        group(0, 2 * gg, True)
        group(1, 2 * gg + 1, False)
        return 0

    lax.fori_loop(0, CPW // NB // 2, body, 0)
    pltpu.make_async_copy(y0_hbm.at[pl.ds(0, NB * CH)], v1, sem_s1).wait()
    pltpu.make_async_copy(earr.at[pl.ds(0, NB)], i0, sem_i0).wait()
    plsc.subcore_barrier()

    pltpu.sync_copy(agg_sh.at[pl.ds(r0, RPT)], zbuf)
    pltpu.sync_copy(zbuf, agg0_out.at[pl.ds(c * NP + r0, RPT)])


_l0 = functools.partial(
    pl.kernel,
    out_type=jax.ShapeDtypeStruct((NCORE * NP,), jnp.float32),
    mesh=_MESH,
    scratch_types=[
        pltpu.VMEM_SHARED((NP,), jnp.float32),
        pltpu.VMEM((NB, 2, CH), jnp.int32),
        pltpu.VMEM((NB, 2, CH), jnp.int32),
        pltpu.VMEM((NB * CH,), jnp.float32),
        pltpu.VMEM((NB * CH,), jnp.float32),
        pltpu.VMEM((RPT,), jnp.float32),
        pltpu.SemaphoreType.DMA,
        pltpu.SemaphoreType.DMA,
        pltpu.SemaphoreType.DMA,
        pltpu.SemaphoreType.DMA,
        pltpu.SemaphoreType.DMA,
        pltpu.SemaphoreType.DMA,
    ],
    compiler_params=_SC_PARAMS,
    interpret=_INTERP,
)(_l0_body)


def _conv_body(earr, x_hbm, z16_hbm,
               agg_out,
               agg_sh, i0, i1, v0, v1, zbuf,
               sem_i0, sem_i1, sem_g0, sem_g1, sem_s0, sem_s1):
    c = lax.axis_index("c")
    s = lax.axis_index("s")
    r0 = s * RPT
    coff = c * NP
    base = s * CPT
    ib = (i0, i1)
    vb = (v0, v1)
    sem_i = (sem_i0, sem_i1)
    sem_g = (sem_g0, sem_g1)
    sem_s = (sem_s0, sem_s1)

    pltpu.sync_copy(z16_hbm, zbuf)
    for r in range(NZ):
        pltpu.async_copy(zbuf, agg_sh.at[pl.ds(r0 + r * ZR, ZR)], sem_g0)
    for r in range(NZ):
        pltpu.make_async_copy(zbuf, agg_sh.at[pl.ds(r0 + r * ZR, ZR)],
                              sem_g0).wait()
    plsc.subcore_barrier()

    pltpu.async_copy(earr.at[pl.ds(base, NB)], i0, sem_i0)

    def group(p, g, first):
        pltpu.make_async_copy(earr.at[pl.ds(0, NB)], ib[p], sem_i[p]).wait()
        if first:
            @pl.when(g >= 1)
            def _():
                pltpu.make_async_copy(x_hbm.at[pl.ds(0, NB * CH)],
                                      vb[1 - p], sem_s[1 - p]).wait()
        else:
            pltpu.make_async_copy(x_hbm.at[pl.ds(0, NB * CH)],
                                  vb[1 - p], sem_s[1 - p]).wait()
        pltpu.async_copy(earr.at[pl.ds(base + NB * (g + 1), NB)],
                         ib[1 - p], sem_i[1 - p])
        for b in range(NB):
            for j in range(CH // 16):
                ib[p][b, 0, pl.ds(j * 16, 16)] = (
                    ib[p][b, 0, pl.ds(j * 16, 16)] + coff)
        for b in range(NB):
            pltpu.async_copy(x_hbm.at[ib[p].at[b, 0]],
                             vb[p].at[pl.ds(b * CH, CH)], sem_g[p])
        pltpu.make_async_copy(x_hbm.at[pl.ds(0, NB * CH)],
                              vb[p], sem_g[p]).wait()
        for b in range(NB):
            pltpu.async_copy(vb[p].at[pl.ds(b * CH, CH)],
                             agg_sh.at[ib[p].at[b, 1]], sem_s[p], add=True)

    def body(gg, _):
        group(0, 2 * gg, True)
        group(1, 2 * gg + 1, False)
        return 0

    lax.fori_loop(0, CPT // NB // 2, body, 0)
    pltpu.make_async_copy(x_hbm.at[pl.ds(0, NB * CH)], v1, sem_s1).wait()
    pltpu.make_async_copy(earr.at[pl.ds(0, NB)], i0, sem_i0).wait()
    plsc.subcore_barrier()

    for r in range(NZ):
        if r % 2 == 0:
            buf = zbuf
            semx = sem_g0
        else:
            buf = v1
            semx = sem_g1
        if r >= 2:
            pltpu.make_async_copy(x_hbm.at[pl.ds(0, ZR)],
                                  buf if r % 2 == 0 else buf.at[pl.ds(0, ZR)],
                                  semx).wait()
        piece = pl.ds(r0 + r * ZR, ZR)
        if r % 2 == 0:
            pltpu.sync_copy(agg_sh.at[piece], buf)
            pltpu.async_copy(buf, agg_out.at[pl.ds(coff + r0 + r * ZR, ZR)],
                             semx)
        else:
            pltpu.sync_copy(agg_sh.at[piece], buf.at[pl.ds(0, ZR)])
            pltpu.async_copy(buf.at[pl.ds(0, ZR)],
                             agg_out.at[pl.ds(coff + r0 + r * ZR, ZR)], semx)
    pltpu.make_async_copy(x_hbm.at[pl.ds(0, ZR)], zbuf, sem_g0).wait()
    pltpu.make_async_copy(x_hbm.at[pl.ds(0, ZR)], v1.at[pl.ds(0, ZR)],
                          sem_g1).wait()


_conv = functools.partial(
    pl.kernel,
    out_type=jax.ShapeDtypeStruct((NCORE * NP, HHID), jnp.float32),
    mesh=_MESH,
    scratch_types=[
        pltpu.VMEM_SHARED((NP, HHID), jnp.float32),
        pltpu.VMEM((NB, 2, CH), jnp.int32),
        pltpu.VMEM((NB, 2, CH), jnp.int32),
        pltpu.VMEM((NB * CH, HHID), jnp.float32),
        pltpu.VMEM((NB * CH, HHID), jnp.float32),
        pltpu.VMEM((ZR, HHID), jnp.float32),
        pltpu.SemaphoreType.DMA,
        pltpu.SemaphoreType.DMA,
        pltpu.SemaphoreType.DMA,
        pltpu.SemaphoreType.DMA,
        pltpu.SemaphoreType.DMA,
        pltpu.SemaphoreType.DMA,
    ],
    compiler_params=_SC_PARAMS,
    interpret=_INTERP,
)(_conv_body)


_RB = RPT
_G = NP // _RB

_col1 = pl.BlockSpec((_RB, 1), lambda i: (i, 0))
_col1b = pl.BlockSpec((_RB, 1), lambda i: (i + _G, 0))
_colH = pl.BlockSpec((_RB, HID), lambda i: (i, 0))
_colHH = pl.BlockSpec((_RB, HHID), lambda i: (i, 0))
_colHHb = pl.BlockSpec((_RB, HHID), lambda i: (i + _G, 0))
_wfull = pl.BlockSpec((HID, HID), lambda i: (0, 0))
_w0full = pl.BlockSpec((1, HID), lambda i: (0, 0))


def _norm_body(oda_ref, odb_ref, ida_ref, idb_ref, l_ref,
               ns_ref, nd_ref, y0_ref):
    od = oda_ref[...] + odb_ref[...]
    idg = ida_ref[...] + idb_ref[...]
    ns = jnp.where(od > 0, lax.rsqrt(jnp.maximum(od, 1.0)), 0.0)
    nd = jnp.where(idg > 0, lax.rsqrt(jnp.maximum(idg, 1.0)), 0.0)
    ns_ref[...] = ns
    nd_ref[...] = nd
    y0_ref[...] = l_ref[...] * ns


_norms = pl.pallas_call(
    _norm_body,
    grid=(_G,),
    in_specs=[_col1, _col1b, _col1, _col1b, _col1],
    out_specs=[_col1, _col1, _col1],
    out_shape=[jax.ShapeDtypeStruct((NP, 1), jnp.float32)] * 3,
    interpret=_INTERP,
)


_col1m = pl.BlockSpec((_RB, 1), lambda i: (i % _G, 0))
_col1bm = pl.BlockSpec((_RB, 1), lambda i: (i % _G + _G, 0))
_colHHm = pl.BlockSpec((_RB, HHID), lambda i: (i % _G, 0))
_colHHbm = pl.BlockSpec((_RB, HHID), lambda i: (i % _G + _G, 0))
_wfullm = pl.BlockSpec((HID, HID), lambda i: (0, 0))
_w0fullm = pl.BlockSpec((1, HID), lambda i: (0, 0))
_colHHout = pl.BlockSpec((_RB, HHID), lambda i: (i, 0))


def _dense0_body(a_ref, a2_ref, nd_ref, ns_ref, w_ref, b_ref, x_ref):
    a = a_ref[...] + a2_ref[...]
    h = a * nd_ref[...]
    out = h * w_ref[...] + b_ref[...]
    out = jnp.maximum(out, 0.0) * ns_ref[...]
    half0 = pl.program_id(0) < _G
    x_ref[...] = jnp.where(half0, out[:, :HHID], out[:, HHID:])


_dense0 = pl.pallas_call(
    _dense0_body,
    grid=(2 * _G,),
    in_specs=[_col1m, _col1bm, _col1m, _col1m, _w0fullm, _w0fullm],
    out_specs=_colHHout,
    out_shape=jax.ShapeDtypeStruct((NCORE * NP, HHID), jnp.float32),
    interpret=_INTERP,
)


def _dense_body(aa_ref, ab_ref, nd_ref, ns_ref, w_ref, b_ref, x_ref):
    agg = jnp.concatenate([aa_ref[...], ab_ref[...]], axis=1)
    h = jnp.dot(agg, w_ref[...], preferred_element_type=jnp.float32)
    out = jnp.maximum(h * nd_ref[...] + b_ref[...], 0.0) * ns_ref[...]
    half0 = pl.program_id(0) < _G
    x_ref[...] = jnp.where(half0, out[:, :HHID], out[:, HHID:])


_dense = pl.pallas_call(
    _dense_body,
    grid=(2 * _G,),
    in_specs=[_colHHm, _colHHbm, _col1m, _col1m, _wfullm, _w0fullm],
    out_specs=_colHHout,
    out_shape=jax.ShapeDtypeStruct((NCORE * NP, HHID), jnp.float32),
    interpret=_INTERP,
)


def _final_body(aa_ref, ab_ref, nd_ref, w_ref, b_ref, out_ref):
    agg = jnp.concatenate([aa_ref[...], ab_ref[...]], axis=1)
    h = jnp.dot(agg, w_ref[...], preferred_element_type=jnp.float32)
    out_ref[...] = h * nd_ref[...] + b_ref[...]


_final = pl.pallas_call(
    _final_body,
    grid=(_G,),
    in_specs=[_colHH, _colHHb, _col1, _wfull, _w0full],
    out_specs=_colH,
    out_shape=jax.ShapeDtypeStruct((NP, HID), jnp.float32),
    interpret=_INTERP,
)


def kernel(l_data, edge_index, W0, b0, W1, b1, W2, b2):
    pad = jnp.full((E2 - E,), NP - 1, jnp.int32)
    srcp = jnp.concatenate([edge_index[0], pad]).reshape(NCH, 1, CH)
    dstp = jnp.concatenate([edge_index[1], pad]).reshape(NCH, 1, CH)
    earr = jnp.concatenate([srcp, dstp], axis=1)
    earr = jnp.pad(earr, ((0, NB), (0, 0), (0, 0)))
    l_pad = jnp.pad(l_data, ((0, NP - N), (0, 0)))

    z1 = jnp.zeros((RPT,), jnp.float32)
    z16 = jnp.zeros((ZR, HHID), jnp.float32)
    ones = jnp.ones((2 * NB, CH), jnp.float32)

    odp, idp = _hist(earr, z1, ones)
    odp = odp.reshape(NCORE * NP, 1)
    idp = idp.reshape(NCORE * NP, 1)
    ns, nd, y0 = _norms(odp, odp, idp, idp, l_pad)

    agg0 = _l0(earr, y0.reshape(NP), z1).reshape(NCORE * NP, 1)

    x = _dense0(agg0, agg0, nd, ns, W0.reshape(1, HID), b0.reshape(1, HID))
    agg = _conv(earr, x, z16)
    x = _dense(agg, agg, nd, ns, W1, b1.reshape(1, HID))
    agg = _conv(earr, x, z16)
    out = _final(agg, agg, nd, W2, b2.reshape(1, HID))
    return out[:N]


# speedup vs baseline: 18.1299x; 1.1699x over previous
import functools

import jax
import jax.numpy as jnp
from jax import lax
from jax.experimental import pallas as pl
from jax.experimental.pallas import tpu as pltpu
from jax.experimental.pallas import tpu_sc as plsc

N = 100000
E = 1600000
HID = 32
HHID = HID // 2
NSUB = 16
NCORE = 2
NW = NCORE * NSUB
NP = 102400
RPT = NP // NSUB
CH = 128
NB = 4
NCH = 12544
E2 = NCH * CH
CPT = NCH // NSUB
CPW = NCH // NW
ZR = 400
NZ = RPT // ZR

_MESH = plsc.VectorSubcoreMesh(core_axis_name="c", subcore_axis_name="s")
_SC_PARAMS = pltpu.CompilerParams(use_tc_tiling_on_sc=False)
_INTERP = False


def _hist_body(earr, z1_hbm, ones_hbm,
               od_out, id_out,
               od_sh, id_sh, i0, i1, onesv, zbuf,
               sem_i0, sem_i1, sem_s0, sem_s1):
    c = lax.axis_index("c")
    s = lax.axis_index("s")
    r0 = s * RPT
    w = c * NSUB + s
    base = w * CPW
    ib = (i0, i1)
    sem_i = (sem_i0, sem_i1)
    sem_s = (sem_s0, sem_s1)

    pltpu.sync_copy(ones_hbm, onesv)
    pltpu.sync_copy(z1_hbm, zbuf)
    pltpu.sync_copy(zbuf, od_sh.at[pl.ds(r0, RPT)])
    pltpu.sync_copy(zbuf, id_sh.at[pl.ds(r0, RPT)])
    plsc.subcore_barrier()

    pltpu.async_copy(earr.at[pl.ds(base, NB)], i0, sem_i0)

    def group(p, g, first):
        ip = sem_i[p]
        pltpu.make_async_copy(earr.at[pl.ds(0, NB)], ib[p], ip).wait()
        if first:
            @pl.when(g >= 1)
            def _():
                pltpu.make_async_copy(ones_hbm, onesv, sem_s[1 - p]).wait()
        else:
            pltpu.make_async_copy(ones_hbm, onesv, sem_s[1 - p]).wait()
        pltpu.async_copy(earr.at[pl.ds(base + NB * (g + 1), NB)],
                         ib[1 - p], sem_i[1 - p])
        for b in range(NB):
            pltpu.async_copy(onesv.at[0], od_sh.at[ib[p].at[b, 0]],
                             sem_s[p], add=True)
            pltpu.async_copy(onesv.at[0], id_sh.at[ib[p].at[b, 1]],
                             sem_s[p], add=True)

    def body(gg, _):
        group(0, 2 * gg, True)
        group(1, 2 * gg + 1, False)
        return 0

    lax.fori_loop(0, CPW // NB // 2, body, 0)
    pltpu.make_async_copy(ones_hbm, onesv, sem_s1).wait()
    pltpu.make_async_copy(earr.at[pl.ds(0, NB)], i0, sem_i0).wait()
    plsc.subcore_barrier()

    o0 = c * NP + r0
    pltpu.sync_copy(od_sh.at[pl.ds(r0, RPT)], zbuf)
    pltpu.sync_copy(zbuf, od_out.at[pl.ds(o0, RPT)])
    pltpu.sync_copy(id_sh.at[pl.ds(r0, RPT)], zbuf)
    pltpu.sync_copy(zbuf, id_out.at[pl.ds(o0, RPT)])


_hist = functools.partial(
    pl.kernel,
    out_type=(jax.ShapeDtypeStruct((NCORE * NP,), jnp.float32),
              jax.ShapeDtypeStruct((NCORE * NP,), jnp.float32)),
    mesh=_MESH,
    scratch_types=[
        pltpu.VMEM_SHARED((NP,), jnp.float32),
        pltpu.VMEM_SHARED((NP,), jnp.float32),
        pltpu.VMEM((NB, 2, CH), jnp.int32),
        pltpu.VMEM((NB, 2, CH), jnp.int32),
        pltpu.VMEM((2 * NB, CH), jnp.float32),
        pltpu.VMEM((RPT,), jnp.float32),
        pltpu.SemaphoreType.DMA,
        pltpu.SemaphoreType.DMA,
        pltpu.SemaphoreType.DMA,
        pltpu.SemaphoreType.DMA,
    ],
    compiler_params=_SC_PARAMS,
    interpret=_INTERP,
)(_hist_body)


def _l0_body(earr, y0_hbm, z1_hbm,
             agg0_out,
             agg_sh, i0, i1, v0, v1, zbuf,
             sem_i0, sem_i1, sem_g0, sem_g1, sem_s0, sem_s1):
    c = lax.axis_index("c")
    s = lax.axis_index("s")
    r0 = s * RPT
    w = c * NSUB + s
    base = w * CPW
    ib = (i0, i1)
    vb = (v0, v1)
    sem_i = (sem_i0, sem_i1)
    sem_g = (sem_g0, sem_g1)
    sem_s = (sem_s0, sem_s1)

    pltpu.sync_copy(z1_hbm, zbuf)
    pltpu.sync_copy(zbuf, agg_sh.at[pl.ds(r0, RPT)])
    plsc.subcore_barrier()

    pltpu.async_copy(earr.at[pl.ds(base, NB)], i0, sem_i0)

    def group(p, g, first):
        pltpu.make_async_copy(earr.at[pl.ds(0, NB)], ib[p], sem_i[p]).wait()
        if first:
            @pl.when(g >= 1)
            def _():
                pltpu.make_async_copy(y0_hbm.at[pl.ds(0, NB * CH)],
                                      vb[1 - p], sem_s[1 - p]).wait()
        else:
            pltpu.make_async_copy(y0_hbm.at[pl.ds(0, NB * CH)],
                                  vb[1 - p], sem_s[1 - p]).wait()
        pltpu.async_copy(earr.at[pl.ds(base + NB * (g + 1), NB)],
                         ib[1 - p], sem_i[1 - p])
        for b in range(NB):
            pltpu.async_copy(y0_hbm.at[ib[p].at[b, 0]],
                             vb[p].at[pl.ds(b * CH, CH)], sem_g[p])
        pltpu.make_async_copy(y0_hbm.at[pl.ds(0, NB * CH)],
                              vb[p], sem_g[p]).wait()
        for b in range(NB):
            pltpu.async_copy(vb[p].at[pl.ds(b * CH, CH)],
                             agg_sh.at[ib[p].at[b, 1]], sem_s[p], add=True)

    def body(gg, _):
        group(0, 2 * gg, True)
        group(1, 2 * gg + 1, False)
        return 0

    lax.fori_loop(0, CPW // NB // 2, body, 0)
    pltpu.make_async_copy(y0_hbm.at[pl.ds(0, NB * CH)], v1, sem_s1).wait()
    pltpu.make_async_copy(earr.at[pl.ds(0, NB)], i0, sem_i0).wait()
    plsc.subcore_barrier()

    pltpu.sync_copy(agg_sh.at[pl.ds(r0, RPT)], zbuf)
    pltpu.sync_copy(zbuf, agg0_out.at[pl.ds(c * NP + r0, RPT)])


_l0 = functools.partial(
    pl.kernel,
    out_type=jax.ShapeDtypeStruct((NCORE * NP,), jnp.float32),
    mesh=_MESH,
    scratch_types=[
        pltpu.VMEM_SHARED((NP,), jnp.float32),
        pltpu.VMEM((NB, 2, CH), jnp.int32),
        pltpu.VMEM((NB, 2, CH), jnp.int32),
        pltpu.VMEM((NB * CH,), jnp.float32),
        pltpu.VMEM((NB * CH,), jnp.float32),
        pltpu.VMEM((RPT,), jnp.float32),
        pltpu.SemaphoreType.DMA,
        pltpu.SemaphoreType.DMA,
        pltpu.SemaphoreType.DMA,
        pltpu.SemaphoreType.DMA,
        pltpu.SemaphoreType.DMA,
        pltpu.SemaphoreType.DMA,
    ],
    compiler_params=_SC_PARAMS,
    interpret=_INTERP,
)(_l0_body)


def _conv_body(earr, x_hbm, z16_hbm,
               agg_out,
               agg_sh, i0, i1, i2, i3, v0, v1, zbuf,
               sem_i0, sem_i1, sem_g0, sem_g1, sem_s0, sem_s1):
    c = lax.axis_index("c")
    s = lax.axis_index("s")
    r0 = s * RPT
    coff = c * NP
    base = s * CPT
    ib = (i0, i1, i2, i3)
    vb = (v0, v1)
    sem_i = (sem_i0, sem_i1)
    sem_g = (sem_g0, sem_g1)
    sem_s = (sem_s0, sem_s1)
    NG = CPT // NB

    pltpu.sync_copy(z16_hbm, zbuf)
    for r in range(NZ):
        pltpu.async_copy(zbuf, agg_sh.at[pl.ds(r0 + r * ZR, ZR)], sem_g0)
    for r in range(NZ):
        pltpu.make_async_copy(zbuf, agg_sh.at[pl.ds(r0 + r * ZR, ZR)],
                              sem_g0).wait()
    plsc.subcore_barrier()

    def adjust(bank):
        for b in range(NB):
            for j in range(CH // 16):
                bank[b, 0, pl.ds(j * 16, 16)] = (
                    bank[b, 0, pl.ds(j * 16, 16)] + coff)

    def gathers(g, bank, vals, sem):
        for b in range(NB):
            pltpu.async_copy(x_hbm.at[bank.at[b, 0]],
                             vals.at[pl.ds(b * CH, CH)], sem)

    pltpu.async_copy(earr.at[pl.ds(base, NB)], i0, sem_i0)
    pltpu.async_copy(earr.at[pl.ds(base + NB, NB)], i1, sem_i1)
    pltpu.make_async_copy(earr.at[pl.ds(0, NB)], i0, sem_i0).wait()
    adjust(i0)
    gathers(0, i0, v0, sem_g0)

    def group(t, j):
        g = 4 * t + j
        q = j
        p = j % 2
        pltpu.make_async_copy(earr.at[pl.ds(0, NB)], ib[(q + 1) % 4],
                              sem_i[1 - p]).wait()
        adjust(ib[(q + 1) % 4])

        def drain_s():
            pltpu.make_async_copy(x_hbm.at[pl.ds(0, NB * CH)],
                                  vb[1 - p], sem_s[1 - p]).wait()
        if j == 0:
            @pl.when(t > 0)
            def _():
                drain_s()
        else:
            drain_s()
        gathers(g + 1, ib[(q + 1) % 4], vb[1 - p], sem_g[1 - p])
        pltpu.async_copy(earr.at[pl.ds(base + NB * (g + 2), NB)],
                         ib[(q + 2) % 4], sem_i[p])
        pltpu.make_async_copy(x_hbm.at[pl.ds(0, NB * CH)], vb[p],
                              sem_g[p]).wait()
        for b in range(NB):
            pltpu.async_copy(vb[p].at[pl.ds(b * CH, CH)],
                             agg_sh.at[ib[q].at[b, 1]], sem_s[p], add=True)

    def body(t, _):
        for j in range(4):
            group(t, j)
        return 0

    lax.fori_loop(0, NG // 4, body, 0)
    pltpu.make_async_copy(x_hbm.at[pl.ds(0, NB * CH)], v1, sem_s1).wait()
    pltpu.make_async_copy(x_hbm.at[pl.ds(0, NB * CH)], v0, sem_g0).wait()
    pltpu.make_async_copy(earr.at[pl.ds(0, NB)], i3, sem_i1).wait()
    plsc.subcore_barrier()

    for r in range(NZ):
        if r % 2 == 0:
            buf = zbuf
            semx = sem_g0
        else:
            buf = v1
            semx = sem_g1
        if r >= 2:
            pltpu.make_async_copy(x_hbm.at[pl.ds(0, ZR)],
                                  buf if r % 2 == 0 else buf.at[pl.ds(0, ZR)],
                                  semx).wait()
        piece = pl.ds(r0 + r * ZR, ZR)
        if r % 2 == 0:
            pltpu.sync_copy(agg_sh.at[piece], buf)
            pltpu.async_copy(buf, agg_out.at[pl.ds(coff + r0 + r * ZR, ZR)],
                             semx)
        else:
            pltpu.sync_copy(agg_sh.at[piece], buf.at[pl.ds(0, ZR)])
            pltpu.async_copy(buf.at[pl.ds(0, ZR)],
                             agg_out.at[pl.ds(coff + r0 + r * ZR, ZR)], semx)
    pltpu.make_async_copy(x_hbm.at[pl.ds(0, ZR)], zbuf, sem_g0).wait()
    pltpu.make_async_copy(x_hbm.at[pl.ds(0, ZR)], v1.at[pl.ds(0, ZR)],
                          sem_g1).wait()


_conv = functools.partial(
    pl.kernel,
    out_type=jax.ShapeDtypeStruct((NCORE * NP, HHID), jnp.float32),
    mesh=_MESH,
    scratch_types=[
        pltpu.VMEM_SHARED((NP, HHID), jnp.float32),
        pltpu.VMEM((NB, 2, CH), jnp.int32),
        pltpu.VMEM((NB, 2, CH), jnp.int32),
        pltpu.VMEM((NB, 2, CH), jnp.int32),
        pltpu.VMEM((NB, 2, CH), jnp.int32),
        pltpu.VMEM((NB * CH, HHID), jnp.float32),
        pltpu.VMEM((NB * CH, HHID), jnp.float32),
        pltpu.VMEM((ZR, HHID), jnp.float32),
        pltpu.SemaphoreType.DMA,
        pltpu.SemaphoreType.DMA,
        pltpu.SemaphoreType.DMA,
        pltpu.SemaphoreType.DMA,
        pltpu.SemaphoreType.DMA,
        pltpu.SemaphoreType.DMA,
    ],
    compiler_params=_SC_PARAMS,
    interpret=_INTERP,
)(_conv_body)


_RB = RPT
_G = NP // _RB

_col1 = pl.BlockSpec((_RB, 1), lambda i: (i, 0))
_col1b = pl.BlockSpec((_RB, 1), lambda i: (i + _G, 0))
_colH = pl.BlockSpec((_RB, HID), lambda i: (i, 0))
_colHH = pl.BlockSpec((_RB, HHID), lambda i: (i, 0))
_colHHb = pl.BlockSpec((_RB, HHID), lambda i: (i + _G, 0))
_wfull = pl.BlockSpec((HID, HID), lambda i: (0, 0))
_w0full = pl.BlockSpec((1, HID), lambda i: (0, 0))


def _norm_body(oda_ref, odb_ref, ida_ref, idb_ref, l_ref,
               ns_ref, nd_ref, y0_ref):
    od = oda_ref[...] + odb_ref[...]
    idg = ida_ref[...] + idb_ref[...]
    ns = jnp.where(od > 0, lax.rsqrt(jnp.maximum(od, 1.0)), 0.0)
    nd = jnp.where(idg > 0, lax.rsqrt(jnp.maximum(idg, 1.0)), 0.0)
    ns_ref[...] = ns
    nd_ref[...] = nd
    y0_ref[...] = l_ref[...] * ns


_norms = pl.pallas_call(
    _norm_body,
    grid=(_G,),
    in_specs=[_col1, _col1b, _col1, _col1b, _col1],
    out_specs=[_col1, _col1, _col1],
    out_shape=[jax.ShapeDtypeStruct((NP, 1), jnp.float32)] * 3,
    interpret=_INTERP,
)


_col1m = pl.BlockSpec((_RB, 1), lambda i: (i % _G, 0))
_col1bm = pl.BlockSpec((_RB, 1), lambda i: (i % _G + _G, 0))
_colHHm = pl.BlockSpec((_RB, HHID), lambda i: (i % _G, 0))
_colHHbm = pl.BlockSpec((_RB, HHID), lambda i: (i % _G + _G, 0))
_wfullm = pl.BlockSpec((HID, HID), lambda i: (0, 0))
_w0fullm = pl.BlockSpec((1, HID), lambda i: (0, 0))
_colHHout = pl.BlockSpec((_RB, HHID), lambda i: (i, 0))


def _dense0_body(a_ref, a2_ref, nd_ref, ns_ref, w_ref, b_ref, x_ref):
    a = a_ref[...] + a2_ref[...]
    h = a * nd_ref[...]
    out = h * w_ref[...] + b_ref[...]
    out = jnp.maximum(out, 0.0) * ns_ref[...]
    half0 = pl.program_id(0) < _G
    x_ref[...] = jnp.where(half0, out[:, :HHID], out[:, HHID:])


_dense0 = pl.pallas_call(
    _dense0_body,
    grid=(2 * _G,),
    in_specs=[_col1m, _col1bm, _col1m, _col1m, _w0fullm, _w0fullm],
    out_specs=_colHHout,
    out_shape=jax.ShapeDtypeStruct((NCORE * NP, HHID), jnp.float32),
    interpret=_INTERP,
)


def _dense_body(aa_ref, ab_ref, nd_ref, ns_ref, w_ref, b_ref, x_ref):
    agg = jnp.concatenate([aa_ref[...], ab_ref[...]], axis=1)
    h = jnp.dot(agg, w_ref[...], preferred_element_type=jnp.float32)
    out = jnp.maximum(h * nd_ref[...] + b_ref[...], 0.0) * ns_ref[...]
    half0 = pl.program_id(0) < _G
    x_ref[...] = jnp.where(half0, out[:, :HHID], out[:, HHID:])


_dense = pl.pallas_call(
    _dense_body,
    grid=(2 * _G,),
    in_specs=[_colHHm, _colHHbm, _col1m, _col1m, _wfullm, _w0fullm],
    out_specs=_colHHout,
    out_shape=jax.ShapeDtypeStruct((NCORE * NP, HHID), jnp.float32),
    interpret=_INTERP,
)


def _final_body(aa_ref, ab_ref, nd_ref, w_ref, b_ref, out_ref):
    agg = jnp.concatenate([aa_ref[...], ab_ref[...]], axis=1)
    h = jnp.dot(agg, w_ref[...], preferred_element_type=jnp.float32)
    out_ref[...] = h * nd_ref[...] + b_ref[...]


_final = pl.pallas_call(
    _final_body,
    grid=(_G,),
    in_specs=[_colHH, _colHHb, _col1, _wfull, _w0full],
    out_specs=_colH,
    out_shape=jax.ShapeDtypeStruct((NP, HID), jnp.float32),
    interpret=_INTERP,
)


def kernel(l_data, edge_index, W0, b0, W1, b1, W2, b2):
    pad = jnp.full((E2 - E,), NP - 1, jnp.int32)
    srcp = jnp.concatenate([edge_index[0], pad]).reshape(NCH, 1, CH)
    dstp = jnp.concatenate([edge_index[1], pad]).reshape(NCH, 1, CH)
    earr = jnp.concatenate([srcp, dstp], axis=1)
    earr = jnp.pad(earr, ((0, 2 * NB), (0, 0), (0, 0)))
    l_pad = jnp.pad(l_data, ((0, NP - N), (0, 0)))

    z1 = jnp.zeros((RPT,), jnp.float32)
    z16 = jnp.zeros((ZR, HHID), jnp.float32)
    ones = jnp.ones((2 * NB, CH), jnp.float32)

    odp, idp = _hist(earr, z1, ones)
    odp = odp.reshape(NCORE * NP, 1)
    idp = idp.reshape(NCORE * NP, 1)
    ns, nd, y0 = _norms(odp, odp, idp, idp, l_pad)

    agg0 = _l0(earr, y0.reshape(NP), z1).reshape(NCORE * NP, 1)

    x = _dense0(agg0, agg0, nd, ns, W0.reshape(1, HID), b0.reshape(1, HID))
    agg = _conv(earr, x, z16)
    x = _dense(agg, agg, nd, ns, W1, b1.reshape(1, HID))
    agg = _conv(earr, x, z16)
    out = _final(agg, agg, nd, W2, b2.reshape(1, HID))
    return out[:N]


# speedup vs baseline: 19.1729x; 1.0575x over previous
import functools

import jax
import jax.numpy as jnp
from jax import lax
from jax.experimental import pallas as pl
from jax.experimental.pallas import tpu as pltpu
from jax.experimental.pallas import tpu_sc as plsc

N = 100000
E = 1600000
HID = 32
HHID = HID // 2
NSUB = 16
NCORE = 2
NW = NCORE * NSUB
NP = 102400
RPT = NP // NSUB
CH = 128
NB = 4
NCH = 12544
E2 = NCH * CH
CPT = NCH // NSUB
CPW = NCH // NW
NB2 = 14
ZR = 400
NZ = RPT // ZR

_MESH = plsc.VectorSubcoreMesh(core_axis_name="c", subcore_axis_name="s")
_SC_PARAMS = pltpu.CompilerParams(use_tc_tiling_on_sc=False)
_INTERP = False


def _hist_body(earr, z1_hbm, ones_hbm,
               od_out, id_out,
               od_sh, id_sh, i0, i1, i2, i3, onesv, zbuf,
               sem_i0, sem_i1, sem_s0, sem_s1):
    c = lax.axis_index("c")
    s = lax.axis_index("s")
    r0 = s * RPT
    w = c * NSUB + s
    base = w * CPW
    ib = (i0, i1, i2, i3)
    sem_i = (sem_i0, sem_i1)
    sem_s = (sem_s0, sem_s1)

    pltpu.sync_copy(ones_hbm, onesv)
    pltpu.sync_copy(z1_hbm, zbuf)
    pltpu.sync_copy(zbuf, od_sh.at[pl.ds(r0, RPT)])
    pltpu.sync_copy(zbuf, id_sh.at[pl.ds(r0, RPT)])
    plsc.subcore_barrier()

    pltpu.async_copy(earr.at[pl.ds(base, NB2)], i0, sem_i0)
    pltpu.async_copy(earr.at[pl.ds(base + NB2, NB2)], i1, sem_i1)
    pltpu.make_async_copy(earr.at[pl.ds(0, NB2)], i0, sem_i0).wait()

    def group(t, j):
        g = 4 * t + j
        q = j
        p = j % 2
        pltpu.make_async_copy(earr.at[pl.ds(0, NB2)], ib[(q + 1) % 4],
                              sem_i[1 - p]).wait()

        def drain_s():
            pltpu.make_async_copy(ones_hbm, onesv, sem_s[1 - p]).wait()
        if j == 0:
            @pl.when(t > 0)
            def _():
                drain_s()
        else:
            drain_s()
        pltpu.async_copy(earr.at[pl.ds(base + NB2 * (g + 2), NB2)],
                         ib[(q + 2) % 4], sem_i[p])
        for b in range(NB2):
            pltpu.async_copy(onesv.at[0], od_sh.at[ib[q].at[b, 0]],
                             sem_s[p], add=True)
            pltpu.async_copy(onesv.at[0], id_sh.at[ib[q].at[b, 1]],
                             sem_s[p], add=True)

    def body(t, _):
        for j in range(4):
            group(t, j)
        return 0

    lax.fori_loop(0, CPW // NB2 // 4, body, 0)
    pltpu.make_async_copy(ones_hbm, onesv, sem_s1).wait()
    pltpu.make_async_copy(earr.at[pl.ds(0, NB2)], i0, sem_i1).wait()
    plsc.subcore_barrier()

    o0 = c * NP + r0
    pltpu.sync_copy(od_sh.at[pl.ds(r0, RPT)], zbuf)
    pltpu.sync_copy(zbuf, od_out.at[pl.ds(o0, RPT)])
    pltpu.sync_copy(id_sh.at[pl.ds(r0, RPT)], zbuf)
    pltpu.sync_copy(zbuf, id_out.at[pl.ds(o0, RPT)])


_hist = functools.partial(
    pl.kernel,
    out_type=(jax.ShapeDtypeStruct((NCORE * NP,), jnp.float32),
              jax.ShapeDtypeStruct((NCORE * NP,), jnp.float32)),
    mesh=_MESH,
    scratch_types=[
        pltpu.VMEM_SHARED((NP,), jnp.float32),
        pltpu.VMEM_SHARED((NP,), jnp.float32),
        pltpu.VMEM((NB2, 2, CH), jnp.int32),
        pltpu.VMEM((NB2, 2, CH), jnp.int32),
        pltpu.VMEM((NB2, 2, CH), jnp.int32),
        pltpu.VMEM((NB2, 2, CH), jnp.int32),
        pltpu.VMEM((2 * NB2, CH), jnp.float32),
        pltpu.VMEM((RPT,), jnp.float32),
        pltpu.SemaphoreType.DMA,
        pltpu.SemaphoreType.DMA,
        pltpu.SemaphoreType.DMA,
        pltpu.SemaphoreType.DMA,
    ],
    compiler_params=_SC_PARAMS,
    interpret=_INTERP,
)(_hist_body)


def _l0_body(earr, y0_hbm, z1_hbm,
             agg0_out,
             agg_sh, i0, i1, i2, i3, v0, v1, zbuf,
             sem_i0, sem_i1, sem_g0, sem_g1, sem_s0, sem_s1):
    c = lax.axis_index("c")
    s = lax.axis_index("s")
    r0 = s * RPT
    w = c * NSUB + s
    base = w * CPW
    ib = (i0, i1, i2, i3)
    vb = (v0, v1)
    sem_i = (sem_i0, sem_i1)
    sem_g = (sem_g0, sem_g1)
    sem_s = (sem_s0, sem_s1)

    pltpu.sync_copy(z1_hbm, zbuf)
    pltpu.sync_copy(zbuf, agg_sh.at[pl.ds(r0, RPT)])
    plsc.subcore_barrier()

    def gathers(bank, vals, sem):
        for b in range(NB2):
            pltpu.async_copy(y0_hbm.at[bank.at[b, 0]],
                             vals.at[pl.ds(b * CH, CH)], sem)

    pltpu.async_copy(earr.at[pl.ds(base, NB2)], i0, sem_i0)
    pltpu.async_copy(earr.at[pl.ds(base + NB2, NB2)], i1, sem_i1)
    pltpu.make_async_copy(earr.at[pl.ds(0, NB2)], i0, sem_i0).wait()
    gathers(i0, v0, sem_g0)

    def group(t, j):
        g = 4 * t + j
        q = j
        p = j % 2
        pltpu.make_async_copy(earr.at[pl.ds(0, NB2)], ib[(q + 1) % 4],
                              sem_i[1 - p]).wait()

        def drain_s():
            pltpu.make_async_copy(y0_hbm.at[pl.ds(0, NB2 * CH)],
                                  vb[1 - p], sem_s[1 - p]).wait()
        if j == 0:
            @pl.when(t > 0)
            def _():
                drain_s()
        else:
            drain_s()
        gathers(ib[(q + 1) % 4], vb[1 - p], sem_g[1 - p])
        pltpu.async_copy(earr.at[pl.ds(base + NB2 * (g + 2), NB2)],
                         ib[(q + 2) % 4], sem_i[p])
        pltpu.make_async_copy(y0_hbm.at[pl.ds(0, NB2 * CH)], vb[p],
                              sem_g[p]).wait()
        for b in range(NB2):
            pltpu.async_copy(vb[p].at[pl.ds(b * CH, CH)],
                             agg_sh.at[ib[q].at[b, 1]], sem_s[p], add=True)

    def body(t, _):
        for j in range(4):
            group(t, j)
        return 0

    lax.fori_loop(0, CPW // NB2 // 4, body, 0)
    pltpu.make_async_copy(y0_hbm.at[pl.ds(0, NB2 * CH)], v1, sem_s1).wait()
    pltpu.make_async_copy(y0_hbm.at[pl.ds(0, NB2 * CH)], v0, sem_g0).wait()
    pltpu.make_async_copy(earr.at[pl.ds(0, NB2)], i3, sem_i1).wait()
    plsc.subcore_barrier()

    pltpu.sync_copy(agg_sh.at[pl.ds(r0, RPT)], zbuf)
    pltpu.sync_copy(zbuf, agg0_out.at[pl.ds(c * NP + r0, RPT)])


_l0 = functools.partial(
    pl.kernel,
    out_type=jax.ShapeDtypeStruct((NCORE * NP,), jnp.float32),
    mesh=_MESH,
    scratch_types=[
        pltpu.VMEM_SHARED((NP,), jnp.float32),
        pltpu.VMEM((NB2, 2, CH), jnp.int32),
        pltpu.VMEM((NB2, 2, CH), jnp.int32),
        pltpu.VMEM((NB2, 2, CH), jnp.int32),
        pltpu.VMEM((NB2, 2, CH), jnp.int32),
        pltpu.VMEM((NB2 * CH,), jnp.float32),
        pltpu.VMEM((NB2 * CH,), jnp.float32),
        pltpu.VMEM((RPT,), jnp.float32),
        pltpu.SemaphoreType.DMA,
        pltpu.SemaphoreType.DMA,
        pltpu.SemaphoreType.DMA,
        pltpu.SemaphoreType.DMA,
        pltpu.SemaphoreType.DMA,
        pltpu.SemaphoreType.DMA,
    ],
    compiler_params=_SC_PARAMS,
    interpret=_INTERP,
)(_l0_body)


def _conv_body(earr, x_hbm, z16_hbm,
               agg_out,
               agg_sh, i0, i1, i2, i3, v0, v1, zbuf,
               sem_i0, sem_i1, sem_g0, sem_g1, sem_s0, sem_s1):
    c = lax.axis_index("c")
    s = lax.axis_index("s")
    r0 = s * RPT
    coff = c * NP
    base = s * CPT
    ib = (i0, i1, i2, i3)
    vb = (v0, v1)
    sem_i = (sem_i0, sem_i1)
    sem_g = (sem_g0, sem_g1)
    sem_s = (sem_s0, sem_s1)
    NG = CPT // NB

    pltpu.sync_copy(z16_hbm, zbuf)
    for r in range(NZ):
        pltpu.async_copy(zbuf, agg_sh.at[pl.ds(r0 + r * ZR, ZR)], sem_g0)
    for r in range(NZ):
        pltpu.make_async_copy(zbuf, agg_sh.at[pl.ds(r0 + r * ZR, ZR)],
                              sem_g0).wait()
    plsc.subcore_barrier()

    def adjust(bank):
        for b in range(NB):
            for j in range(CH // 16):
                bank[b, 0, pl.ds(j * 16, 16)] = (
                    bank[b, 0, pl.ds(j * 16, 16)] + coff)

    def gathers(g, bank, vals, sem):
        for b in range(NB):
            pltpu.async_copy(x_hbm.at[bank.at[b, 0]],
                             vals.at[pl.ds(b * CH, CH)], sem)

    pltpu.async_copy(earr.at[pl.ds(base, NB)], i0, sem_i0)
    pltpu.async_copy(earr.at[pl.ds(base + NB, NB)], i1, sem_i1)
    pltpu.make_async_copy(earr.at[pl.ds(0, NB)], i0, sem_i0).wait()
    adjust(i0)
    gathers(0, i0, v0, sem_g0)

    def group(t, j):
        g = 4 * t + j
        q = j
        p = j % 2
        pltpu.make_async_copy(earr.at[pl.ds(0, NB)], ib[(q + 1) % 4],
                              sem_i[1 - p]).wait()
        adjust(ib[(q + 1) % 4])

        def drain_s():
            pltpu.make_async_copy(x_hbm.at[pl.ds(0, NB * CH)],
                                  vb[1 - p], sem_s[1 - p]).wait()
        if j == 0:
            @pl.when(t > 0)
            def _():
                drain_s()
        else:
            drain_s()
        gathers(g + 1, ib[(q + 1) % 4], vb[1 - p], sem_g[1 - p])
        pltpu.async_copy(earr.at[pl.ds(base + NB * (g + 2), NB)],
                         ib[(q + 2) % 4], sem_i[p])
        pltpu.make_async_copy(x_hbm.at[pl.ds(0, NB * CH)], vb[p],
                              sem_g[p]).wait()
        for b in range(NB):
            pltpu.async_copy(vb[p].at[pl.ds(b * CH, CH)],
                             agg_sh.at[ib[q].at[b, 1]], sem_s[p], add=True)

    def body(t, _):
        for j in range(4):
            group(t, j)
        return 0

    lax.fori_loop(0, NG // 4, body, 0)
    pltpu.make_async_copy(x_hbm.at[pl.ds(0, NB * CH)], v1, sem_s1).wait()
    pltpu.make_async_copy(x_hbm.at[pl.ds(0, NB * CH)], v0, sem_g0).wait()
    pltpu.make_async_copy(earr.at[pl.ds(0, NB)], i3, sem_i1).wait()
    plsc.subcore_barrier()

    for r in range(NZ):
        if r % 2 == 0:
            buf = zbuf
            semx = sem_g0
        else:
            buf = v1
            semx = sem_g1
        if r >= 2:
            pltpu.make_async_copy(x_hbm.at[pl.ds(0, ZR)],
                                  buf if r % 2 == 0 else buf.at[pl.ds(0, ZR)],
                                  semx).wait()
        piece = pl.ds(r0 + r * ZR, ZR)
        if r % 2 == 0:
            pltpu.sync_copy(agg_sh.at[piece], buf)
            pltpu.async_copy(buf, agg_out.at[pl.ds(coff + r0 + r * ZR, ZR)],
                             semx)
        else:
            pltpu.sync_copy(agg_sh.at[piece], buf.at[pl.ds(0, ZR)])
            pltpu.async_copy(buf.at[pl.ds(0, ZR)],
                             agg_out.at[pl.ds(coff + r0 + r * ZR, ZR)], semx)
    pltpu.make_async_copy(x_hbm.at[pl.ds(0, ZR)], zbuf, sem_g0).wait()
    pltpu.make_async_copy(x_hbm.at[pl.ds(0, ZR)], v1.at[pl.ds(0, ZR)],
                          sem_g1).wait()


_conv = functools.partial(
    pl.kernel,
    out_type=jax.ShapeDtypeStruct((NCORE * NP, HHID), jnp.float32),
    mesh=_MESH,
    scratch_types=[
        pltpu.VMEM_SHARED((NP, HHID), jnp.float32),
        pltpu.VMEM((NB, 2, CH), jnp.int32),
        pltpu.VMEM((NB, 2, CH), jnp.int32),
        pltpu.VMEM((NB, 2, CH), jnp.int32),
        pltpu.VMEM((NB, 2, CH), jnp.int32),
        pltpu.VMEM((NB * CH, HHID), jnp.float32),
        pltpu.VMEM((NB * CH, HHID), jnp.float32),
        pltpu.VMEM((ZR, HHID), jnp.float32),
        pltpu.SemaphoreType.DMA,
        pltpu.SemaphoreType.DMA,
        pltpu.SemaphoreType.DMA,
        pltpu.SemaphoreType.DMA,
        pltpu.SemaphoreType.DMA,
        pltpu.SemaphoreType.DMA,
    ],
    compiler_params=_SC_PARAMS,
    interpret=_INTERP,
)(_conv_body)


_RB = RPT
_G = NP // _RB

_col1 = pl.BlockSpec((_RB, 1), lambda i: (i, 0))
_col1b = pl.BlockSpec((_RB, 1), lambda i: (i + _G, 0))
_colH = pl.BlockSpec((_RB, HID), lambda i: (i, 0))
_colHH = pl.BlockSpec((_RB, HHID), lambda i: (i, 0))
_colHHb = pl.BlockSpec((_RB, HHID), lambda i: (i + _G, 0))
_wfull = pl.BlockSpec((HID, HID), lambda i: (0, 0))
_w0full = pl.BlockSpec((1, HID), lambda i: (0, 0))


def _norm_body(oda_ref, odb_ref, ida_ref, idb_ref, l_ref,
               ns_ref, nd_ref, y0_ref):
    od = oda_ref[...] + odb_ref[...]
    idg = ida_ref[...] + idb_ref[...]
    ns = jnp.where(od > 0, lax.rsqrt(jnp.maximum(od, 1.0)), 0.0)
    nd = jnp.where(idg > 0, lax.rsqrt(jnp.maximum(idg, 1.0)), 0.0)
    ns_ref[...] = ns
    nd_ref[...] = nd
    y0_ref[...] = l_ref[...] * ns


_norms = pl.pallas_call(
    _norm_body,
    grid=(_G,),
    in_specs=[_col1, _col1b, _col1, _col1b, _col1],
    out_specs=[_col1, _col1, _col1],
    out_shape=[jax.ShapeDtypeStruct((NP, 1), jnp.float32)] * 3,
    interpret=_INTERP,
)


_col1m = pl.BlockSpec((_RB, 1), lambda i: (i % _G, 0))
_col1bm = pl.BlockSpec((_RB, 1), lambda i: (i % _G + _G, 0))
_colHHm = pl.BlockSpec((_RB, HHID), lambda i: (i % _G, 0))
_colHHbm = pl.BlockSpec((_RB, HHID), lambda i: (i % _G + _G, 0))
_wfullm = pl.BlockSpec((HID, HID), lambda i: (0, 0))
_w0fullm = pl.BlockSpec((1, HID), lambda i: (0, 0))
_colHHout = pl.BlockSpec((_RB, HHID), lambda i: (i, 0))


def _dense0_body(a_ref, a2_ref, nd_ref, ns_ref, w_ref, b_ref, x_ref):
    a = a_ref[...] + a2_ref[...]
    h = a * nd_ref[...]
    out = h * w_ref[...] + b_ref[...]
    out = jnp.maximum(out, 0.0) * ns_ref[...]
    half0 = pl.program_id(0) < _G
    x_ref[...] = jnp.where(half0, out[:, :HHID], out[:, HHID:])


_dense0 = pl.pallas_call(
    _dense0_body,
    grid=(2 * _G,),
    in_specs=[_col1m, _col1bm, _col1m, _col1m, _w0fullm, _w0fullm],
    out_specs=_colHHout,
    out_shape=jax.ShapeDtypeStruct((NCORE * NP, HHID), jnp.float32),
    interpret=_INTERP,
)


def _dense_body(aa_ref, ab_ref, nd_ref, ns_ref, w_ref, b_ref, x_ref):
    agg = jnp.concatenate([aa_ref[...], ab_ref[...]], axis=1)
    h = jnp.dot(agg, w_ref[...], preferred_element_type=jnp.float32)
    out = jnp.maximum(h * nd_ref[...] + b_ref[...], 0.0) * ns_ref[...]
    half0 = pl.program_id(0) < _G
    x_ref[...] = jnp.where(half0, out[:, :HHID], out[:, HHID:])


_dense = pl.pallas_call(
    _dense_body,
    grid=(2 * _G,),
    in_specs=[_colHHm, _colHHbm, _col1m, _col1m, _wfullm, _w0fullm],
    out_specs=_colHHout,
    out_shape=jax.ShapeDtypeStruct((NCORE * NP, HHID), jnp.float32),
    interpret=_INTERP,
)


def _final_body(aa_ref, ab_ref, nd_ref, w_ref, b_ref, out_ref):
    agg = jnp.concatenate([aa_ref[...], ab_ref[...]], axis=1)
    h = jnp.dot(agg, w_ref[...], preferred_element_type=jnp.float32)
    out_ref[...] = h * nd_ref[...] + b_ref[...]


_final = pl.pallas_call(
    _final_body,
    grid=(_G,),
    in_specs=[_colHH, _colHHb, _col1, _wfull, _w0full],
    out_specs=_colH,
    out_shape=jax.ShapeDtypeStruct((NP, HID), jnp.float32),
    interpret=_INTERP,
)


def kernel(l_data, edge_index, W0, b0, W1, b1, W2, b2):
    pad = jnp.full((E2 - E,), NP - 1, jnp.int32)
    srcp = jnp.concatenate([edge_index[0], pad]).reshape(NCH, 1, CH)
    dstp = jnp.concatenate([edge_index[1], pad]).reshape(NCH, 1, CH)
    earr = jnp.concatenate([srcp, dstp], axis=1)
    earr = jnp.pad(earr, ((0, 2 * NB2), (0, 0), (0, 0)))
    l_pad = jnp.pad(l_data, ((0, NP - N), (0, 0)))

    z1 = jnp.zeros((RPT,), jnp.float32)
    z16 = jnp.zeros((ZR, HHID), jnp.float32)
    ones = jnp.ones((2 * NB2, CH), jnp.float32)

    odp, idp = _hist(earr, z1, ones)
    odp = odp.reshape(NCORE * NP, 1)
    idp = idp.reshape(NCORE * NP, 1)
    ns, nd, y0 = _norms(odp, odp, idp, idp, l_pad)

    agg0 = _l0(earr, y0.reshape(NP), z1).reshape(NCORE * NP, 1)

    x = _dense0(agg0, agg0, nd, ns, W0.reshape(1, HID), b0.reshape(1, HID))
    agg = _conv(earr, x, z16)
    x = _dense(agg, agg, nd, ns, W1, b1.reshape(1, HID))
    agg = _conv(earr, x, z16)
    out = _final(agg, agg, nd, W2, b2.reshape(1, HID))
    return out[:N]


# speedup vs baseline: 19.1857x; 1.0007x over previous
import functools

import jax
import jax.numpy as jnp
from jax import lax
from jax.experimental import pallas as pl
from jax.experimental.pallas import tpu as pltpu
from jax.experimental.pallas import tpu_sc as plsc

N = 100000
E = 1600000
HID = 32
HHID = HID // 2
NSUB = 16
NCORE = 2
NW = NCORE * NSUB
NP = 102400
RPT = NP // NSUB
CH = 128
NB = 4
NCH = 12544
E2 = NCH * CH
CPT = NCH // NSUB
CPW = NCH // NW
NB2 = 14
ZR = 400
NZ = RPT // ZR

_MESH = plsc.VectorSubcoreMesh(core_axis_name="c", subcore_axis_name="s")
_SC_PARAMS = pltpu.CompilerParams(use_tc_tiling_on_sc=False)


def _hist_body(earr, z1_hbm, ones_hbm,
               od_out, id_out,
               od_sh, id_sh, i0, i1, i2, i3, onesv, zbuf,
               sem_i0, sem_i1, sem_s0, sem_s1):
    c = lax.axis_index("c")
    s = lax.axis_index("s")
    r0 = s * RPT
    w = c * NSUB + s
    base = w * CPW
    ib = (i0, i1, i2, i3)
    sem_i = (sem_i0, sem_i1)
    sem_s = (sem_s0, sem_s1)

    pltpu.sync_copy(ones_hbm, onesv)
    pltpu.sync_copy(z1_hbm, zbuf)
    pltpu.sync_copy(zbuf, od_sh.at[pl.ds(r0, RPT)])
    pltpu.sync_copy(zbuf, id_sh.at[pl.ds(r0, RPT)])
    plsc.subcore_barrier()

    pltpu.async_copy(earr.at[pl.ds(base, NB2)], i0, sem_i0)
    pltpu.async_copy(earr.at[pl.ds(base + NB2, NB2)], i1, sem_i1)
    pltpu.make_async_copy(earr.at[pl.ds(0, NB2)], i0, sem_i0).wait()

    def group(t, j):
        g = 4 * t + j
        q = j
        p = j % 2
        pltpu.make_async_copy(earr.at[pl.ds(0, NB2)], ib[(q + 1) % 4],
                              sem_i[1 - p]).wait()

        def drain_s():
            pltpu.make_async_copy(ones_hbm, onesv, sem_s[1 - p]).wait()
        if j == 0:
            @pl.when(t > 0)
            def _():
                drain_s()
        else:
            drain_s()
        pltpu.async_copy(earr.at[pl.ds(base + NB2 * (g + 2), NB2)],
                         ib[(q + 2) % 4], sem_i[p])
        for b in range(NB2):
            pltpu.async_copy(onesv.at[0], od_sh.at[ib[q].at[b, 0]],
                             sem_s[p], add=True)
            pltpu.async_copy(onesv.at[0], id_sh.at[ib[q].at[b, 1]],
                             sem_s[p], add=True)

    def body(t, _):
        for j in range(4):
            group(t, j)
        return 0

    lax.fori_loop(0, CPW // NB2 // 4, body, 0)
    pltpu.make_async_copy(ones_hbm, onesv, sem_s1).wait()
    pltpu.make_async_copy(earr.at[pl.ds(0, NB2)], i0, sem_i1).wait()
    plsc.subcore_barrier()

    o0 = c * NP + r0
    pltpu.sync_copy(od_sh.at[pl.ds(r0, RPT)], zbuf)
    pltpu.sync_copy(zbuf, od_out.at[pl.ds(o0, RPT)])
    pltpu.sync_copy(id_sh.at[pl.ds(r0, RPT)], zbuf)
    pltpu.sync_copy(zbuf, id_out.at[pl.ds(o0, RPT)])


_hist = functools.partial(
    pl.kernel,
    out_type=(jax.ShapeDtypeStruct((NCORE * NP,), jnp.float32),
              jax.ShapeDtypeStruct((NCORE * NP,), jnp.float32)),
    mesh=_MESH,
    scratch_types=[
        pltpu.VMEM_SHARED((NP,), jnp.float32),
        pltpu.VMEM_SHARED((NP,), jnp.float32),
        pltpu.VMEM((NB2, 2, CH), jnp.int32),
        pltpu.VMEM((NB2, 2, CH), jnp.int32),
        pltpu.VMEM((NB2, 2, CH), jnp.int32),
        pltpu.VMEM((NB2, 2, CH), jnp.int32),
        pltpu.VMEM((2 * NB2, CH), jnp.float32),
        pltpu.VMEM((RPT,), jnp.float32),
        pltpu.SemaphoreType.DMA,
        pltpu.SemaphoreType.DMA,
        pltpu.SemaphoreType.DMA,
        pltpu.SemaphoreType.DMA,
    ],
    compiler_params=_SC_PARAMS,
)(_hist_body)


def _l0_body(earr, y0_hbm, z1_hbm,
             agg0_out,
             agg_sh, i0, i1, i2, i3, v0, v1, zbuf,
             sem_i0, sem_i1, sem_g0, sem_g1, sem_s0, sem_s1):
    c = lax.axis_index("c")
    s = lax.axis_index("s")
    r0 = s * RPT
    w = c * NSUB + s
    base = w * CPW
    ib = (i0, i1, i2, i3)
    vb = (v0, v1)
    sem_i = (sem_i0, sem_i1)
    sem_g = (sem_g0, sem_g1)
    sem_s = (sem_s0, sem_s1)

    pltpu.sync_copy(z1_hbm, zbuf)
    pltpu.sync_copy(zbuf, agg_sh.at[pl.ds(r0, RPT)])
    plsc.subcore_barrier()

    def gathers(bank, vals, sem):
        for b in range(NB2):
            pltpu.async_copy(y0_hbm.at[bank.at[b, 0]],
                             vals.at[pl.ds(b * CH, CH)], sem)

    pltpu.async_copy(earr.at[pl.ds(base, NB2)], i0, sem_i0)
    pltpu.async_copy(earr.at[pl.ds(base + NB2, NB2)], i1, sem_i1)
    pltpu.make_async_copy(earr.at[pl.ds(0, NB2)], i0, sem_i0).wait()
    gathers(i0, v0, sem_g0)

    def group(t, j):
        g = 4 * t + j
        q = j
        p = j % 2
        pltpu.make_async_copy(earr.at[pl.ds(0, NB2)], ib[(q + 1) % 4],
                              sem_i[1 - p]).wait()

        def drain_s():
            pltpu.make_async_copy(y0_hbm.at[pl.ds(0, NB2 * CH)],
                                  vb[1 - p], sem_s[1 - p]).wait()
        if j == 0:
            @pl.when(t > 0)
            def _():
                drain_s()
        else:
            drain_s()
        gathers(ib[(q + 1) % 4], vb[1 - p], sem_g[1 - p])
        pltpu.async_copy(earr.at[pl.ds(base + NB2 * (g + 2), NB2)],
                         ib[(q + 2) % 4], sem_i[p])
        pltpu.make_async_copy(y0_hbm.at[pl.ds(0, NB2 * CH)], vb[p],
                              sem_g[p]).wait()
        for b in range(NB2):
            pltpu.async_copy(vb[p].at[pl.ds(b * CH, CH)],
                             agg_sh.at[ib[q].at[b, 1]], sem_s[p], add=True)

    def body(t, _):
        for j in range(4):
            group(t, j)
        return 0

    lax.fori_loop(0, CPW // NB2 // 4, body, 0)
    pltpu.make_async_copy(y0_hbm.at[pl.ds(0, NB2 * CH)], v1, sem_s1).wait()
    pltpu.make_async_copy(y0_hbm.at[pl.ds(0, NB2 * CH)], v0, sem_g0).wait()
    pltpu.make_async_copy(earr.at[pl.ds(0, NB2)], i3, sem_i1).wait()
    plsc.subcore_barrier()

    pltpu.sync_copy(agg_sh.at[pl.ds(r0, RPT)], zbuf)
    pltpu.sync_copy(zbuf, agg0_out.at[pl.ds(c * NP + r0, RPT)])


_l0 = functools.partial(
    pl.kernel,
    out_type=jax.ShapeDtypeStruct((NCORE * NP,), jnp.float32),
    mesh=_MESH,
    scratch_types=[
        pltpu.VMEM_SHARED((NP,), jnp.float32),
        pltpu.VMEM((NB2, 2, CH), jnp.int32),
        pltpu.VMEM((NB2, 2, CH), jnp.int32),
        pltpu.VMEM((NB2, 2, CH), jnp.int32),
        pltpu.VMEM((NB2, 2, CH), jnp.int32),
        pltpu.VMEM((NB2 * CH,), jnp.float32),
        pltpu.VMEM((NB2 * CH,), jnp.float32),
        pltpu.VMEM((RPT,), jnp.float32),
        pltpu.SemaphoreType.DMA,
        pltpu.SemaphoreType.DMA,
        pltpu.SemaphoreType.DMA,
        pltpu.SemaphoreType.DMA,
        pltpu.SemaphoreType.DMA,
        pltpu.SemaphoreType.DMA,
    ],
    compiler_params=_SC_PARAMS,
)(_l0_body)


def _conv_body(earr, x_hbm, z16_hbm,
               agg_out,
               agg_sh, i0, i1, i2, i3, v0, v1, zbuf,
               sem_i0, sem_i1, sem_g0, sem_g1, sem_s0, sem_s1):
    c = lax.axis_index("c")
    s = lax.axis_index("s")
    r0 = s * RPT
    coff = c * NP
    base = s * CPT
    ib = (i0, i1, i2, i3)
    vb = (v0, v1)
    sem_i = (sem_i0, sem_i1)
    sem_g = (sem_g0, sem_g1)
    sem_s = (sem_s0, sem_s1)
    NG = CPT // NB

    pltpu.sync_copy(z16_hbm, zbuf)
    for r in range(NZ):
        pltpu.async_copy(zbuf, agg_sh.at[pl.ds(r0 + r * ZR, ZR)], sem_g0)
    for r in range(NZ):
        pltpu.make_async_copy(zbuf, agg_sh.at[pl.ds(r0 + r * ZR, ZR)],
                              sem_g0).wait()
    plsc.subcore_barrier()

    def adjust(bank):
        for b in range(NB):
            for j in range(CH // 16):
                bank[b, 0, pl.ds(j * 16, 16)] = (
                    bank[b, 0, pl.ds(j * 16, 16)] + coff)

    def gathers(g, bank, vals, sem):
        for b in range(NB):
            pltpu.async_copy(x_hbm.at[bank.at[b, 0]],
                             vals.at[pl.ds(b * CH, CH)], sem)

    pltpu.async_copy(earr.at[pl.ds(base, NB)], i0, sem_i0)
    pltpu.async_copy(earr.at[pl.ds(base + NB, NB)], i1, sem_i1)
    pltpu.make_async_copy(earr.at[pl.ds(0, NB)], i0, sem_i0).wait()
    adjust(i0)
    gathers(0, i0, v0, sem_g0)

    def group(t, j):
        g = 4 * t + j
        q = j
        p = j % 2
        pltpu.make_async_copy(earr.at[pl.ds(0, NB)], ib[(q + 1) % 4],
                              sem_i[1 - p]).wait()
        adjust(ib[(q + 1) % 4])

        def drain_s():
            pltpu.make_async_copy(x_hbm.at[pl.ds(0, NB * CH)],
                                  vb[1 - p], sem_s[1 - p]).wait()
        if j == 0:
            @pl.when(t > 0)
            def _():
                drain_s()
        else:
            drain_s()
        gathers(g + 1, ib[(q + 1) % 4], vb[1 - p], sem_g[1 - p])
        pltpu.async_copy(earr.at[pl.ds(base + NB * (g + 2), NB)],
                         ib[(q + 2) % 4], sem_i[p])
        pltpu.make_async_copy(x_hbm.at[pl.ds(0, NB * CH)], vb[p],
                              sem_g[p]).wait()
        for b in range(NB):
            pltpu.async_copy(vb[p].at[pl.ds(b * CH, CH)],
                             agg_sh.at[ib[q].at[b, 1]], sem_s[p], add=True)

    def body(t, _):
        for j in range(4):
            group(t, j)
        return 0

    lax.fori_loop(0, NG // 4, body, 0)
    pltpu.make_async_copy(x_hbm.at[pl.ds(0, NB * CH)], v1, sem_s1).wait()
    pltpu.make_async_copy(x_hbm.at[pl.ds(0, NB * CH)], v0, sem_g0).wait()
    pltpu.make_async_copy(earr.at[pl.ds(0, NB)], i3, sem_i1).wait()
    plsc.subcore_barrier()

    for r in range(NZ):
        if r % 2 == 0:
            buf = zbuf
            semx = sem_g0
        else:
            buf = v1
            semx = sem_g1
        if r >= 2:
            pltpu.make_async_copy(x_hbm.at[pl.ds(0, ZR)],
                                  buf if r % 2 == 0 else buf.at[pl.ds(0, ZR)],
                                  semx).wait()
        piece = pl.ds(r0 + r * ZR, ZR)
        if r % 2 == 0:
            pltpu.sync_copy(agg_sh.at[piece], buf)
            pltpu.async_copy(buf, agg_out.at[pl.ds(coff + r0 + r * ZR, ZR)],
                             semx)
        else:
            pltpu.sync_copy(agg_sh.at[piece], buf.at[pl.ds(0, ZR)])
            pltpu.async_copy(buf.at[pl.ds(0, ZR)],
                             agg_out.at[pl.ds(coff + r0 + r * ZR, ZR)], semx)
    pltpu.make_async_copy(x_hbm.at[pl.ds(0, ZR)], zbuf, sem_g0).wait()
    pltpu.make_async_copy(x_hbm.at[pl.ds(0, ZR)], v1.at[pl.ds(0, ZR)],
                          sem_g1).wait()


_conv = functools.partial(
    pl.kernel,
    out_type=jax.ShapeDtypeStruct((NCORE * NP, HHID), jnp.float32),
    mesh=_MESH,
    scratch_types=[
        pltpu.VMEM_SHARED((NP, HHID), jnp.float32),
        pltpu.VMEM((NB, 2, CH), jnp.int32),
        pltpu.VMEM((NB, 2, CH), jnp.int32),
        pltpu.VMEM((NB, 2, CH), jnp.int32),
        pltpu.VMEM((NB, 2, CH), jnp.int32),
        pltpu.VMEM((NB * CH, HHID), jnp.float32),
        pltpu.VMEM((NB * CH, HHID), jnp.float32),
        pltpu.VMEM((ZR, HHID), jnp.float32),
        pltpu.SemaphoreType.DMA,
        pltpu.SemaphoreType.DMA,
        pltpu.SemaphoreType.DMA,
        pltpu.SemaphoreType.DMA,
        pltpu.SemaphoreType.DMA,
        pltpu.SemaphoreType.DMA,
    ],
    compiler_params=_SC_PARAMS,
)(_conv_body)


_RB = RPT
_G = NP // _RB

_col1 = pl.BlockSpec((_RB, 1), lambda i: (i, 0))
_col1b = pl.BlockSpec((_RB, 1), lambda i: (i + _G, 0))
_colH = pl.BlockSpec((_RB, HID), lambda i: (i, 0))
_colHH = pl.BlockSpec((_RB, HHID), lambda i: (i, 0))
_colHHb = pl.BlockSpec((_RB, HHID), lambda i: (i + _G, 0))
_wfull = pl.BlockSpec((HID, HID), lambda i: (0, 0))
_w0full = pl.BlockSpec((1, HID), lambda i: (0, 0))


def _norm_body(oda_ref, odb_ref, ida_ref, idb_ref, l_ref,
               ns_ref, nd_ref, y0_ref):
    od = oda_ref[...] + odb_ref[...]
    idg = ida_ref[...] + idb_ref[...]
    ns = jnp.where(od > 0, lax.rsqrt(jnp.maximum(od, 1.0)), 0.0)
    nd = jnp.where(idg > 0, lax.rsqrt(jnp.maximum(idg, 1.0)), 0.0)
    ns_ref[...] = ns
    nd_ref[...] = nd
    y0_ref[...] = l_ref[...] * ns


_norms = pl.pallas_call(
    _norm_body,
    grid=(_G,),
    in_specs=[_col1, _col1b, _col1, _col1b, _col1],
    out_specs=[_col1, _col1, _col1],
    out_shape=[jax.ShapeDtypeStruct((NP, 1), jnp.float32)] * 3,
)


_col1m = pl.BlockSpec((_RB, 1), lambda i: (i % _G, 0))
_col1bm = pl.BlockSpec((_RB, 1), lambda i: (i % _G + _G, 0))
_colHHm = pl.BlockSpec((_RB, HHID), lambda i: (i % _G, 0))
_colHHbm = pl.BlockSpec((_RB, HHID), lambda i: (i % _G + _G, 0))
_wfullm = pl.BlockSpec((HID, HID), lambda i: (0, 0))
_w0fullm = pl.BlockSpec((1, HID), lambda i: (0, 0))
_colHHout = pl.BlockSpec((_RB, HHID), lambda i: (i, 0))


def _dense0_body(a_ref, a2_ref, nd_ref, ns_ref, w_ref, b_ref, x_ref):
    a = a_ref[...] + a2_ref[...]
    h = a * nd_ref[...]
    out = h * w_ref[...] + b_ref[...]
    out = jnp.maximum(out, 0.0) * ns_ref[...]
    half0 = pl.program_id(0) < _G
    x_ref[...] = jnp.where(half0, out[:, :HHID], out[:, HHID:])


_dense0 = pl.pallas_call(
    _dense0_body,
    grid=(2 * _G,),
    in_specs=[_col1m, _col1bm, _col1m, _col1m, _w0fullm, _w0fullm],
    out_specs=_colHHout,
    out_shape=jax.ShapeDtypeStruct((NCORE * NP, HHID), jnp.float32),
)


def _dense_body(aa_ref, ab_ref, nd_ref, ns_ref, w_ref, b_ref, x_ref):
    agg = jnp.concatenate([aa_ref[...], ab_ref[...]], axis=1)
    h = jnp.dot(agg, w_ref[...], preferred_element_type=jnp.float32)
    out = jnp.maximum(h * nd_ref[...] + b_ref[...], 0.0) * ns_ref[...]
    half0 = pl.program_id(0) < _G
    x_ref[...] = jnp.where(half0, out[:, :HHID], out[:, HHID:])


_dense = pl.pallas_call(
    _dense_body,
    grid=(2 * _G,),
    in_specs=[_colHHm, _colHHbm, _col1m, _col1m, _wfullm, _w0fullm],
    out_specs=_colHHout,
    out_shape=jax.ShapeDtypeStruct((NCORE * NP, HHID), jnp.float32),
)


def _final_body(aa_ref, ab_ref, nd_ref, w_ref, b_ref, out_ref):
    agg = jnp.concatenate([aa_ref[...], ab_ref[...]], axis=1)
    h = jnp.dot(agg, w_ref[...], preferred_element_type=jnp.float32)
    out_ref[...] = h * nd_ref[...] + b_ref[...]


_final = pl.pallas_call(
    _final_body,
    grid=(_G,),
    in_specs=[_colHH, _colHHb, _col1, _wfull, _w0full],
    out_specs=_colH,
    out_shape=jax.ShapeDtypeStruct((NP, HID), jnp.float32),
)


def kernel(l_data, edge_index, W0, b0, W1, b1, W2, b2):
    pad = jnp.full((E2 - E,), NP - 1, jnp.int32)
    srcp = jnp.concatenate([edge_index[0], pad]).reshape(NCH, 1, CH)
    dstp = jnp.concatenate([edge_index[1], pad]).reshape(NCH, 1, CH)
    earr = jnp.concatenate([srcp, dstp], axis=1)
    earr = jnp.pad(earr, ((0, 2 * NB2), (0, 0), (0, 0)))
    l_pad = jnp.pad(l_data, ((0, NP - N), (0, 0)))

    z1 = jnp.zeros((RPT,), jnp.float32)
    z16 = jnp.zeros((ZR, HHID), jnp.float32)
    ones = jnp.ones((2 * NB2, CH), jnp.float32)

    odp, idp = _hist(earr, z1, ones)
    odp = odp.reshape(NCORE * NP, 1)
    idp = idp.reshape(NCORE * NP, 1)
    ns, nd, y0 = _norms(odp, odp, idp, idp, l_pad)

    agg0 = _l0(earr, y0.reshape(NP), z1).reshape(NCORE * NP, 1)

    x = _dense0(agg0, agg0, nd, ns, W0.reshape(1, HID), b0.reshape(1, HID))
    agg = _conv(earr, x, z16)
    x = _dense(agg, agg, nd, ns, W1, b1.reshape(1, HID))
    agg = _conv(earr, x, z16)
    out = _final(agg, agg, nd, W2, b2.reshape(1, HID))
    return out[:N]
